# baseline (speedup 1.0000x reference)
import functools

import jax
import jax.numpy as jnp
from jax import lax
from jax.experimental import pallas as pl
from jax.experimental.pallas import tpu as pltpu
from jax.experimental.pallas import tpu_sc as plsc

N = 50000
E = 800000
T = 4
FEAT = 64
HID = 128
EDGE = 16
B = 1024
XW = T * FEAT

NCORES = 2
NSUB = 16
CHUNK = 4480
NCHUNK = 12
NPASS = NCHUNK // NCORES
NPAD = NCHUNK * CHUNK
EPT = E // NSUB
EB = 2000
NBLK = EPT // EB
SUB = 128
ROWS_PT = CHUNK // NSUB


def _edge_pass(x, src, dst, ea):
    mesh = plsc.VectorSubcoreMesh(core_axis_name="c", subcore_axis_name="s")

    @functools.partial(
        pl.kernel,
        out_type=(
            jax.ShapeDtypeStruct((NPAD, XW), jnp.float32),
            jax.ShapeDtypeStruct((NPAD, EDGE), jnp.float32),
            jax.ShapeDtypeStruct((NPAD, EDGE), jnp.float32),
        ),
        mesh=mesh,
        compiler_params=pltpu.CompilerParams(needs_layout_passes=False, use_tc_tiling_on_sc=False),
        scratch_types=dict(
            accX=pltpu.VMEM_SHARED((CHUNK + 16, XW), jnp.float32),
            accE=pltpu.VMEM_SHARED((CHUNK + 16, EDGE), jnp.float32),
            accC=pltpu.VMEM_SHARED((CHUNK + 16, EDGE), jnp.float32),
            src_blk=pltpu.VMEM((EB,), jnp.int32),
            dst_blk=pltpu.VMEM((EB,), jnp.int32),
            csrc=pltpu.VMEM((17, SUB), jnp.int32),
            cdst=pltpu.VMEM((17, SUB), jnp.int32),
            ceid=pltpu.VMEM((17, SUB), jnp.int32),
            stage_x=pltpu.VMEM((SUB, XW), jnp.float32),
            ea_st=pltpu.VMEM((SUB, EDGE), jnp.float32),
            ones_b=pltpu.VMEM((SUB, EDGE), jnp.float32),
            gsem=pltpu.SemaphoreType.DMA,
            gsem2=pltpu.SemaphoreType.DMA,
        ),
    )
    def edge_kernel(x_hbm, src_hbm, dst_hbm, ea_hbm, a_out, se_out, cn_out,
                    accX, accE, accC, src_blk, dst_blk, csrc, cdst, ceid,
                    stage_x, ea_st, ones_b, gsem, gsem2):
        cid = lax.axis_index("c")
        sid = lax.axis_index("s")

        zero16 = jnp.zeros((16,), jnp.float32)
        one16 = jnp.ones((16,), jnp.float32)

        def init_row(i, _):
            ones_b[i, pl.ds(0, 16)] = one16
            return 0

        lax.fori_loop(0, SUB, init_row, 0)

        def pass_body(p, _):
            lo = (NCORES * p + cid) * CHUNK
            def zrow(i, _):
                for q in range(XW // 16):
                    stage_x[i, pl.ds(q * 16, 16)] = zero16
                ea_st[i, pl.ds(0, 16)] = zero16
                return 0

            lax.fori_loop(0, SUB, zrow, 0)
            r0 = sid * ROWS_PT
            zoff = 0
            for zn in (128, 128, 24):
                pltpu.sync_copy(stage_x.at[pl.ds(0, zn)],
                                accX.at[pl.ds(r0 + zoff, zn)])
                pltpu.sync_copy(ea_st.at[pl.ds(0, zn)],
                                accE.at[pl.ds(r0 + zoff, zn)])
                pltpu.sync_copy(ea_st.at[pl.ds(0, zn)],
                                accC.at[pl.ds(r0 + zoff, zn)])
                zoff += zn
            @pl.when(sid == 0)
            def _():
                pltpu.sync_copy(stage_x.at[pl.ds(0, 16)],
                                accX.at[pl.ds(CHUNK, 16)])
                pltpu.sync_copy(ea_st.at[pl.ds(0, 16)],
                                accE.at[pl.ds(CHUNK, 16)])
                pltpu.sync_copy(ea_st.at[pl.ds(0, 16)],
                                accC.at[pl.ds(CHUNK, 16)])
            plsc.subcore_barrier()

            def blk_body(bk, _):
                base = sid * EPT + bk * EB
                pltpu.sync_copy(src_hbm.at[pl.ds(base, EB)], src_blk)
                pltpu.sync_copy(dst_hbm.at[pl.ds(base, EB)], dst_blk)

                def filt(i, k):
                    d = dst_blk[pl.ds(i * 16, 16)]
                    s = src_blk[pl.ds(i * 16, 16)]
                    m = (d >= lo) & (d < lo + CHUNK)
                    mi = m.astype(jnp.int32)
                    pos = k + plsc.cumsum(mi) - 1
                    pr = lax.shift_right_logical(pos, 7)
                    pc = lax.bitwise_and(pos, 127)
                    plsc.store_scatter(cdst, [pr, pc], d - lo, mask=m)
                    plsc.store_scatter(csrc, [pr, pc], s, mask=m)
                    ev = base + i * 16 + lax.iota(jnp.int32, 16)
                    plsc.store_scatter(ceid, [pr, pc], ev, mask=m)
                    return k + jnp.sum(mi)

                k = lax.fori_loop(0, EB // 16, filt, 0)
                dummy_d = jnp.full((16,), CHUNK, jnp.int32)
                zero_i = jnp.zeros((16,), jnp.int32)
                for q in range(SUB // 16):
                    pos = k + q * 16 + lax.iota(jnp.int32, 16)
                    pr = lax.shift_right_logical(pos, 7)
                    pc = lax.bitwise_and(pos, 127)
                    plsc.store_scatter(cdst, [pr, pc], dummy_d)
                    plsc.store_scatter(csrc, [pr, pc], zero_i)
                    plsc.store_scatter(ceid, [pr, pc], zero_i)
                nsub = (k + SUB - 1) // SUB

                def sub(j, _):
                    cp1 = pltpu.async_copy(
                        x_hbm.at[csrc.at[j]], stage_x, gsem)
                    cp2 = pltpu.async_copy(
                        ea_hbm.at[ceid.at[j]], ea_st, gsem2)
                    cp1.wait()
                    cp2.wait()
                    pltpu.sync_copy(stage_x, accX.at[cdst.at[j]], add=True)
                    pltpu.sync_copy(ea_st, accE.at[cdst.at[j]], add=True)
                    pltpu.sync_copy(ones_b, accC.at[cdst.at[j]], add=True)
                    return 0

                lax.fori_loop(0, nsub, sub, 0)
                return 0

            lax.fori_loop(0, NBLK, blk_body, 0)
            plsc.subcore_barrier()
            pltpu.sync_copy(accX.at[pl.ds(r0, ROWS_PT)],
                            a_out.at[pl.ds(lo + r0, ROWS_PT)])
            pltpu.sync_copy(accE.at[pl.ds(r0, ROWS_PT)],
                            se_out.at[pl.ds(lo + r0, ROWS_PT)])
            pltpu.sync_copy(accC.at[pl.ds(r0, ROWS_PT)],
                            cn_out.at[pl.ds(lo + r0, ROWS_PT)])
            return 0

        lax.fori_loop(0, NPASS, pass_body, 0)

    return edge_kernel(x, src, dst, ea)


def _focal_gather(focal, x, a, se, cn):
    mesh = plsc.VectorSubcoreMesh(core_axis_name="c", subcore_axis_name="s")
    RPW = B // (NCORES * NSUB)

    @functools.partial(
        pl.kernel,
        out_type=(
            jax.ShapeDtypeStruct((B, XW), jnp.float32),
            jax.ShapeDtypeStruct((B, XW), jnp.float32),
            jax.ShapeDtypeStruct((B, EDGE), jnp.float32),
            jax.ShapeDtypeStruct((B, EDGE), jnp.float32),
        ),
        mesh=mesh,
        compiler_params=pltpu.CompilerParams(needs_layout_passes=False, use_tc_tiling_on_sc=False),
        scratch_types=dict(
            pidx=pltpu.VMEM((RPW,), jnp.int32),
            bufx=pltpu.VMEM((RPW, XW), jnp.float32),
            bufa=pltpu.VMEM((RPW, XW), jnp.float32),
            bufs=pltpu.VMEM((RPW, EDGE), jnp.float32),
            bufc=pltpu.VMEM((RPW, EDGE), jnp.float32),
            sem=pltpu.SemaphoreType.DMA,
        ),
    )
    def focal_kernel(f_hbm, x_hbm, a_hbm, se_hbm, cn_hbm,
                     xf, af, sf, cf, pidx, bufx, bufa, bufs, bufc, sem):
        wid = lax.axis_index("s") * NCORES + lax.axis_index("c")
        base = wid * RPW
        pltpu.sync_copy(f_hbm.at[pl.ds(base, RPW)], pidx)
        pltpu.async_copy(x_hbm.at[pidx], bufx, sem).wait()
        pltpu.sync_copy(bufx, xf.at[pl.ds(base, RPW)])
        pltpu.async_copy(a_hbm.at[pidx], bufa, sem).wait()
        pltpu.sync_copy(bufa, af.at[pl.ds(base, RPW)])
        pltpu.async_copy(se_hbm.at[pidx], bufs, sem).wait()
        pltpu.sync_copy(bufs, sf.at[pl.ds(base, RPW)])
        pltpu.async_copy(cn_hbm.at[pidx], bufc, sem).wait()
        pltpu.sync_copy(bufc, cf.at[pl.ds(base, RPW)])

    return focal_kernel(focal, x, a, se, cn)


NB_STATS = 1000
NGRID = N // NB_STATS


def _stats_body(x_ref, a_ref, se_ref, cn_ref, wm_ref, ws_ref, bm_ref, bs_ref,
                o_ref):
    pc = pl.program_id(0)

    @pl.when(pc == 0)
    def _():
        o_ref[...] = jnp.zeros_like(o_ref)

    wx = wm_ref[0:FEAT, :]
    we = wm_ref[FEAT:FEAT + EDGE, :]
    ws = ws_ref[...]
    cn = cn_ref[:, 0:1]
    basem = jnp.dot(se_ref[...], we, preferred_element_type=jnp.float32)
    basem = basem + cn * bm_ref[...]
    rinv = 1.0 / jnp.maximum(cn, 1.0)
    rows = []
    rows2 = []
    for t in range(T):
        xt = x_ref[:, FEAT * t:FEAT * (t + 1)]
        at = a_ref[:, FEAT * t:FEAT * (t + 1)]
        out = jnp.dot(xt, ws, preferred_element_type=jnp.float32)
        out = out + bs_ref[...]
        out = out + (jnp.dot(at, wx, preferred_element_type=jnp.float32)
                     + basem) * rinv
        rows.append(jnp.sum(out, axis=0, keepdims=True))
        rows2.append(jnp.sum(out * out, axis=0, keepdims=True))
    o_ref[...] += jnp.concatenate(rows + rows2, axis=0)


def _stats(x, a, se, cn, w_msg, w_self, b_msg, b_self):
    return pl.pallas_call(
        _stats_body,
        grid=(NGRID,),
        in_specs=[
            pl.BlockSpec((NB_STATS, XW), lambda i: (i, 0)),
            pl.BlockSpec((NB_STATS, XW), lambda i: (i, 0)),
            pl.BlockSpec((NB_STATS, EDGE), lambda i: (i, 0)),
            pl.BlockSpec((NB_STATS, EDGE), lambda i: (i, 0)),
            pl.BlockSpec((FEAT + EDGE, HID), lambda i: (0, 0)),
            pl.BlockSpec((FEAT, HID), lambda i: (0, 0)),
            pl.BlockSpec((1, HID), lambda i: (0, 0)),
            pl.BlockSpec((1, HID), lambda i: (0, 0)),
        ],
        out_specs=pl.BlockSpec((2 * T, HID), lambda i: (0, 0)),
        out_shape=jax.ShapeDtypeStruct((2 * T, HID), jnp.float32),
    )(x, a, se, cn, w_msg, w_self, b_msg, b_self)


def _final_body(xf_ref, af_ref, sf_ref, cf_ref, st_ref, wm_ref, ws_ref,
                bm_ref, bs_ref, g_ref, bb_ref, wih_ref, whh_ref, bih_ref,
                bhh_ref, wcls_ref, bcls_ref, o_ref):
    wx = wm_ref[0:FEAT, :]
    we = wm_ref[FEAT:FEAT + EDGE, :]
    ws = ws_ref[...]
    mean = st_ref[0:T, :] * (1.0 / N)
    var = st_ref[T:2 * T, :] * (1.0 / N) - mean * mean
    scale = g_ref[...] * lax.rsqrt(var + 1e-5)
    shift = bb_ref[...] - mean * scale
    cn = cf_ref[:, 0:1]
    basem = jnp.dot(sf_ref[...], we, preferred_element_type=jnp.float32)
    basem = basem + cn * bm_ref[...]
    rinv = 1.0 / jnp.maximum(cn, 1.0)
    h = jnp.zeros((B, HID), jnp.float32)
    c = jnp.zeros((B, HID), jnp.float32)
    for t in range(T):
        xt = xf_ref[:, FEAT * t:FEAT * (t + 1)]
        at = af_ref[:, FEAT * t:FEAT * (t + 1)]
        out = jnp.dot(xt, ws, preferred_element_type=jnp.float32)
        out = out + bs_ref[...]
        out = out + (jnp.dot(at, wx, preferred_element_type=jnp.float32)
                     + basem) * rinv
        ht = jnp.maximum(out * scale[t:t + 1, :] + shift[t:t + 1, :], 0.0)
        gates = (jnp.dot(ht, wih_ref[...], preferred_element_type=jnp.float32)
                 + bih_ref[...]
                 + jnp.dot(h, whh_ref[...], preferred_element_type=jnp.float32)
                 + bhh_ref[...])
        i_g = jax.nn.sigmoid(gates[:, 0 * HID:1 * HID])
        f_g = jax.nn.sigmoid(gates[:, 1 * HID:2 * HID])
        g_g = jnp.tanh(gates[:, 2 * HID:3 * HID])
        o_g = jax.nn.sigmoid(gates[:, 3 * HID:4 * HID])
        c = f_g * c + i_g * g_g
        h = o_g * jnp.tanh(c)
    o_ref[...] = jnp.dot(h, wcls_ref[...],
                         preferred_element_type=jnp.float32) + bcls_ref[...]


def _final(xf, af, sf, cf, stats, w_msg, w_self, b_msg, b_self, gam, bet,
           w_ih, w_hh, b_ih, b_hh, wcls_p, bcls_p):
    return pl.pallas_call(
        _final_body,
        out_shape=jax.ShapeDtypeStruct((B, HID), jnp.float32),
    )(xf, af, sf, cf, stats, w_msg, w_self, b_msg, b_self, gam, bet,
      w_ih, w_hh, b_ih, b_hh, wcls_p, bcls_p)


def kernel(x, edge_index, edge_attr, ptr, W_msg, b_msg, W_self, b_self,
           bn_gamma, bn_beta, W_ih, W_hh, b_ih, b_hh, W_cls, b_cls):
    src = edge_index[0]
    dst = edge_index[1]
    focal = ptr[:B]

    a, se, cn = _edge_pass(x, src, dst, edge_attr)
    xf, af, sf, cf = _focal_gather(focal, x, a, se, cn)

    b_msg2 = b_msg.reshape(1, HID)
    b_self2 = b_self.reshape(1, HID)
    stats = _stats(x, a, se, cn, W_msg, W_self, b_msg2, b_self2)

    gam = jnp.broadcast_to(bn_gamma.reshape(1, HID), (T, HID))
    bet = jnp.broadcast_to(bn_beta.reshape(1, HID), (T, HID))
    wcls_p = jnp.zeros((HID, HID), jnp.float32).at[:, :2].set(W_cls)
    bcls_p = jnp.zeros((1, HID), jnp.float32).at[0, :2].set(b_cls)
    out = _final(xf, af, sf, cf, stats, W_msg, W_self, b_msg2, b_self2,
                 gam, bet, W_ih, W_hh, b_ih.reshape(1, 4 * HID),
                 b_hh.reshape(1, 4 * HID), wcls_p, bcls_p)
    return out[:, :2]


# speedup vs baseline: 2.8724x; 2.8724x over previous
import functools

import jax
import jax.numpy as jnp
from jax import lax
from jax.experimental import pallas as pl
from jax.experimental.pallas import tpu as pltpu
from jax.experimental.pallas import tpu_sc as plsc

N = 50000
E = 800000
T = 4
FEAT = 64
HID = 128
EDGE = 16
B = 1024
XW = T * FEAT

NCORES = 2
NSUB = 16
CHUNK = 4480
NCHUNK = 12
NPASS = NCHUNK // NCORES
NPAD = NCHUNK * CHUNK
EPT = E // NSUB
EB = 2000
NBLK = EPT // EB
SUB = 64
ROWS_PT = CHUNK // NSUB


def _edge_pass(x, xb, src, dst, ea):
    mesh = plsc.VectorSubcoreMesh(core_axis_name="c", subcore_axis_name="s")

    @functools.partial(
        pl.kernel,
        out_type=(
            jax.ShapeDtypeStruct((NPAD, XW), jnp.float32),
            jax.ShapeDtypeStruct((NPAD, EDGE), jnp.float32),
            jax.ShapeDtypeStruct((NPAD, EDGE), jnp.float32),
        ),
        mesh=mesh,
        compiler_params=pltpu.CompilerParams(needs_layout_passes=False, use_tc_tiling_on_sc=False),
        scratch_types=dict(
            accX=pltpu.VMEM_SHARED((CHUNK + 16, XW), jnp.float32),
            accE=pltpu.VMEM_SHARED((CHUNK + 16, EDGE), jnp.float32),
            accC=pltpu.VMEM_SHARED((CHUNK + 16, EDGE), jnp.float32),
            src_blk=pltpu.VMEM((EB,), jnp.int32),
            dst_blk=pltpu.VMEM((EB,), jnp.int32),
            csrc=pltpu.VMEM((33, SUB), jnp.int32),
            cdst=pltpu.VMEM((33, SUB), jnp.int32),
            ceid=pltpu.VMEM((33, SUB), jnp.int32),
            stage_b=pltpu.VMEM((2, SUB, XW), jnp.bfloat16),
            stage_x=pltpu.VMEM((SUB, XW), jnp.float32),
            ea_st=pltpu.VMEM((2, SUB, EDGE), jnp.float32),
            ones_b=pltpu.VMEM((SUB, EDGE), jnp.float32),
            gsem=pltpu.SemaphoreType.DMA,
            gsem2=pltpu.SemaphoreType.DMA,
        ),
    )
    def edge_kernel(x_hbm, xb_hbm, src_hbm, dst_hbm, ea_hbm, a_out, se_out, cn_out,
                    accX, accE, accC, src_blk, dst_blk, csrc, cdst, ceid,
                    stage_b, stage_x, ea_st, ones_b, gsem, gsem2):
        cid = lax.axis_index("c")
        sid = lax.axis_index("s")

        zero16 = jnp.zeros((16,), jnp.float32)
        one16 = jnp.ones((16,), jnp.float32)

        def init_row(i, _):
            ones_b[i, pl.ds(0, 16)] = one16
            return 0

        lax.fori_loop(0, SUB, init_row, 0)

        def pass_body(p, _):
            lo = (NCORES * p + cid) * CHUNK
            def zrow(i, _):
                for q in range(XW // 16):
                    stage_x[i, pl.ds(q * 16, 16)] = zero16
                ea_st[0, i, pl.ds(0, 16)] = zero16
                return 0

            lax.fori_loop(0, SUB, zrow, 0)
            r0 = sid * ROWS_PT
            zoff = 0
            for zn in (64, 64, 64, 64, 24):
                pltpu.sync_copy(stage_x.at[pl.ds(0, zn)],
                                accX.at[pl.ds(r0 + zoff, zn)])
                pltpu.sync_copy(ea_st.at[0].at[pl.ds(0, zn)],
                                accE.at[pl.ds(r0 + zoff, zn)])
                pltpu.sync_copy(ea_st.at[0].at[pl.ds(0, zn)],
                                accC.at[pl.ds(r0 + zoff, zn)])
                zoff += zn
            @pl.when(sid == 0)
            def _():
                pltpu.sync_copy(stage_x.at[pl.ds(0, 16)],
                                accX.at[pl.ds(CHUNK, 16)])
                pltpu.sync_copy(ea_st.at[0].at[pl.ds(0, 16)],
                                accE.at[pl.ds(CHUNK, 16)])
                pltpu.sync_copy(ea_st.at[0].at[pl.ds(0, 16)],
                                accC.at[pl.ds(CHUNK, 16)])
            plsc.subcore_barrier()

            def blk_body(bk, _):
                base = sid * EPT + bk * EB
                pltpu.sync_copy(src_hbm.at[pl.ds(base, EB)], src_blk)
                pltpu.sync_copy(dst_hbm.at[pl.ds(base, EB)], dst_blk)

                def filt(i, k):
                    d = dst_blk[pl.ds(i * 16, 16)]
                    s = src_blk[pl.ds(i * 16, 16)]
                    m = (d >= lo) & (d < lo + CHUNK)
                    mi = m.astype(jnp.int32)
                    pos = k + plsc.cumsum(mi) - 1
                    pr = lax.shift_right_logical(pos, 6)
                    pc = lax.bitwise_and(pos, 63)
                    plsc.store_scatter(cdst, [pr, pc], d - lo, mask=m)
                    plsc.store_scatter(csrc, [pr, pc], s, mask=m)
                    ev = base + i * 16 + lax.iota(jnp.int32, 16)
                    plsc.store_scatter(ceid, [pr, pc], ev, mask=m)
                    return k + jnp.sum(mi)

                k = lax.fori_loop(0, EB // 16, filt, 0)
                dummy_d = jnp.full((16,), CHUNK, jnp.int32)
                zero_i = jnp.zeros((16,), jnp.int32)
                for q in range(SUB // 16):
                    pos = k + q * 16 + lax.iota(jnp.int32, 16)
                    pr = lax.shift_right_logical(pos, 6)
                    pc = lax.bitwise_and(pos, 63)
                    plsc.store_scatter(cdst, [pr, pc], dummy_d)
                    plsc.store_scatter(csrc, [pr, pc], zero_i)
                    plsc.store_scatter(ceid, [pr, pc], zero_i)
                nsub = (k + SUB - 1) // SUB

                @pl.when(nsub > 0)
                def _():
                    pltpu.async_copy(xb_hbm.at[csrc.at[0]],
                                     stage_b.at[0], gsem)
                    pltpu.async_copy(ea_hbm.at[ceid.at[0]],
                                     ea_st.at[0], gsem2)

                def sub(j, _):
                    s = j & 1
                    pltpu.make_async_copy(xb_hbm.at[csrc.at[j]],
                                          stage_b.at[s], gsem).wait()
                    pltpu.make_async_copy(ea_hbm.at[ceid.at[j]],
                                          ea_st.at[s], gsem2).wait()
                    @pl.when(j + 1 < nsub)
                    def _():
                        pltpu.async_copy(xb_hbm.at[csrc.at[j + 1]],
                                         stage_b.at[1 - s], gsem)
                        pltpu.async_copy(ea_hbm.at[ceid.at[j + 1]],
                                         ea_st.at[1 - s], gsem2)

                    def urow(r, _):
                        for q in range(XW // 32):
                            v = stage_b[s, r, pl.ds(q * 32, 32)]
                            a, b2 = plsc.unpack(
                                v, format=plsc.PackFormat.INTERLEAVED)
                            stage_x[r, pl.ds(q * 32, 16)] = a
                            stage_x[r, pl.ds(q * 32 + 16, 16)] = b2
                        return 0

                    lax.fori_loop(0, SUB, urow, 0)
                    pltpu.sync_copy(stage_x, accX.at[cdst.at[j]],
                                    add=True)
                    pltpu.sync_copy(ea_st.at[s], accE.at[cdst.at[j]],
                                    add=True)
                    pltpu.sync_copy(ones_b, accC.at[cdst.at[j]], add=True)
                    return 0

                lax.fori_loop(0, nsub, sub, 0)
                return 0

            lax.fori_loop(0, NBLK, blk_body, 0)
            plsc.subcore_barrier()
            pltpu.sync_copy(accX.at[pl.ds(r0, ROWS_PT)],
                            a_out.at[pl.ds(lo + r0, ROWS_PT)])
            pltpu.sync_copy(accE.at[pl.ds(r0, ROWS_PT)],
                            se_out.at[pl.ds(lo + r0, ROWS_PT)])
            pltpu.sync_copy(accC.at[pl.ds(r0, ROWS_PT)],
                            cn_out.at[pl.ds(lo + r0, ROWS_PT)])
            return 0

        lax.fori_loop(0, NPASS, pass_body, 0)

    return edge_kernel(x, xb, src, dst, ea)


def _focal_gather(focal, x, a, se, cn):
    mesh = plsc.VectorSubcoreMesh(core_axis_name="c", subcore_axis_name="s")
    RPW = B // (NCORES * NSUB)

    @functools.partial(
        pl.kernel,
        out_type=(
            jax.ShapeDtypeStruct((B, XW), jnp.float32),
            jax.ShapeDtypeStruct((B, XW), jnp.float32),
            jax.ShapeDtypeStruct((B, EDGE), jnp.float32),
            jax.ShapeDtypeStruct((B, EDGE), jnp.float32),
        ),
        mesh=mesh,
        compiler_params=pltpu.CompilerParams(needs_layout_passes=False, use_tc_tiling_on_sc=False),
        scratch_types=dict(
            pidx=pltpu.VMEM((RPW,), jnp.int32),
            bufx=pltpu.VMEM((RPW, XW), jnp.float32),
            bufa=pltpu.VMEM((RPW, XW), jnp.float32),
            bufs=pltpu.VMEM((RPW, EDGE), jnp.float32),
            bufc=pltpu.VMEM((RPW, EDGE), jnp.float32),
            sem=pltpu.SemaphoreType.DMA,
        ),
    )
    def focal_kernel(f_hbm, x_hbm, a_hbm, se_hbm, cn_hbm,
                     xf, af, sf, cf, pidx, bufx, bufa, bufs, bufc, sem):
        wid = lax.axis_index("s") * NCORES + lax.axis_index("c")
        base = wid * RPW
        pltpu.sync_copy(f_hbm.at[pl.ds(base, RPW)], pidx)
        pltpu.async_copy(x_hbm.at[pidx], bufx, sem).wait()
        pltpu.sync_copy(bufx, xf.at[pl.ds(base, RPW)])
        pltpu.async_copy(a_hbm.at[pidx], bufa, sem).wait()
        pltpu.sync_copy(bufa, af.at[pl.ds(base, RPW)])
        pltpu.async_copy(se_hbm.at[pidx], bufs, sem).wait()
        pltpu.sync_copy(bufs, sf.at[pl.ds(base, RPW)])
        pltpu.async_copy(cn_hbm.at[pidx], bufc, sem).wait()
        pltpu.sync_copy(bufc, cf.at[pl.ds(base, RPW)])

    return focal_kernel(focal, x, a, se, cn)


NB_STATS = 1000
NGRID = N // NB_STATS


def _stats_body(x_ref, a_ref, se_ref, cn_ref, wm_ref, ws_ref, bm_ref, bs_ref,
                o_ref):
    pc = pl.program_id(0)

    @pl.when(pc == 0)
    def _():
        o_ref[...] = jnp.zeros_like(o_ref)

    wx = wm_ref[0:FEAT, :]
    we = wm_ref[FEAT:FEAT + EDGE, :]
    ws = ws_ref[...]
    cn = cn_ref[:, 0:1]
    basem = jnp.dot(se_ref[...], we, preferred_element_type=jnp.float32)
    basem = basem + cn * bm_ref[...]
    rinv = 1.0 / jnp.maximum(cn, 1.0)
    rows = []
    rows2 = []
    for t in range(T):
        xt = x_ref[:, FEAT * t:FEAT * (t + 1)]
        at = a_ref[:, FEAT * t:FEAT * (t + 1)]
        out = jnp.dot(xt, ws, preferred_element_type=jnp.float32)
        out = out + bs_ref[...]
        out = out + (jnp.dot(at, wx, preferred_element_type=jnp.float32)
                     + basem) * rinv
        rows.append(jnp.sum(out, axis=0, keepdims=True))
        rows2.append(jnp.sum(out * out, axis=0, keepdims=True))
    o_ref[...] += jnp.concatenate(rows + rows2, axis=0)


def _stats(x, a, se, cn, w_msg, w_self, b_msg, b_self):
    return pl.pallas_call(
        _stats_body,
        grid=(NGRID,),
        in_specs=[
            pl.BlockSpec((NB_STATS, XW), lambda i: (i, 0)),
            pl.BlockSpec((NB_STATS, XW), lambda i: (i, 0)),
            pl.BlockSpec((NB_STATS, EDGE), lambda i: (i, 0)),
            pl.BlockSpec((NB_STATS, EDGE), lambda i: (i, 0)),
            pl.BlockSpec((FEAT + EDGE, HID), lambda i: (0, 0)),
            pl.BlockSpec((FEAT, HID), lambda i: (0, 0)),
            pl.BlockSpec((1, HID), lambda i: (0, 0)),
            pl.BlockSpec((1, HID), lambda i: (0, 0)),
        ],
        out_specs=pl.BlockSpec((2 * T, HID), lambda i: (0, 0)),
        out_shape=jax.ShapeDtypeStruct((2 * T, HID), jnp.float32),
    )(x, a, se, cn, w_msg, w_self, b_msg, b_self)


def _final_body(xf_ref, af_ref, sf_ref, cf_ref, st_ref, wm_ref, ws_ref,
                bm_ref, bs_ref, g_ref, bb_ref, wih_ref, whh_ref, bih_ref,
                bhh_ref, wcls_ref, bcls_ref, o_ref):
    wx = wm_ref[0:FEAT, :]
    we = wm_ref[FEAT:FEAT + EDGE, :]
    ws = ws_ref[...]
    mean = st_ref[0:T, :] * (1.0 / N)
    var = st_ref[T:2 * T, :] * (1.0 / N) - mean * mean
    scale = g_ref[...] * lax.rsqrt(var + 1e-5)
    shift = bb_ref[...] - mean * scale
    cn = cf_ref[:, 0:1]
    basem = jnp.dot(sf_ref[...], we, preferred_element_type=jnp.float32)
    basem = basem + cn * bm_ref[...]
    rinv = 1.0 / jnp.maximum(cn, 1.0)
    h = jnp.zeros((B, HID), jnp.float32)
    c = jnp.zeros((B, HID), jnp.float32)
    for t in range(T):
        xt = xf_ref[:, FEAT * t:FEAT * (t + 1)]
        at = af_ref[:, FEAT * t:FEAT * (t + 1)]
        out = jnp.dot(xt, ws, preferred_element_type=jnp.float32)
        out = out + bs_ref[...]
        out = out + (jnp.dot(at, wx, preferred_element_type=jnp.float32)
                     + basem) * rinv
        ht = jnp.maximum(out * scale[t:t + 1, :] + shift[t:t + 1, :], 0.0)
        gates = (jnp.dot(ht, wih_ref[...], preferred_element_type=jnp.float32)
                 + bih_ref[...]
                 + jnp.dot(h, whh_ref[...], preferred_element_type=jnp.float32)
                 + bhh_ref[...])
        i_g = jax.nn.sigmoid(gates[:, 0 * HID:1 * HID])
        f_g = jax.nn.sigmoid(gates[:, 1 * HID:2 * HID])
        g_g = jnp.tanh(gates[:, 2 * HID:3 * HID])
        o_g = jax.nn.sigmoid(gates[:, 3 * HID:4 * HID])
        c = f_g * c + i_g * g_g
        h = o_g * jnp.tanh(c)
    o_ref[...] = jnp.dot(h, wcls_ref[...],
                         preferred_element_type=jnp.float32) + bcls_ref[...]


def _final(xf, af, sf, cf, stats, w_msg, w_self, b_msg, b_self, gam, bet,
           w_ih, w_hh, b_ih, b_hh, wcls_p, bcls_p):
    return pl.pallas_call(
        _final_body,
        out_shape=jax.ShapeDtypeStruct((B, HID), jnp.float32),
    )(xf, af, sf, cf, stats, w_msg, w_self, b_msg, b_self, gam, bet,
      w_ih, w_hh, b_ih, b_hh, wcls_p, bcls_p)


def kernel(x, edge_index, edge_attr, ptr, W_msg, b_msg, W_self, b_self,
           bn_gamma, bn_beta, W_ih, W_hh, b_ih, b_hh, W_cls, b_cls):
    src = edge_index[0]
    dst = edge_index[1]
    focal = ptr[:B]

    perm = (jnp.arange(XW).reshape(XW // 32, 2, 16)
            .transpose(0, 2, 1).reshape(XW))
    a, se, cn = _edge_pass(x, x[:, perm].astype(jnp.bfloat16), src, dst,
                           edge_attr)
    xf, af, sf, cf = _focal_gather(focal, x, a, se, cn)

    b_msg2 = b_msg.reshape(1, HID)
    b_self2 = b_self.reshape(1, HID)
    stats = _stats(x, a, se, cn, W_msg, W_self, b_msg2, b_self2)

    gam = jnp.broadcast_to(bn_gamma.reshape(1, HID), (T, HID))
    bet = jnp.broadcast_to(bn_beta.reshape(1, HID), (T, HID))
    wcls_p = jnp.zeros((HID, HID), jnp.float32).at[:, :2].set(W_cls)
    bcls_p = jnp.zeros((1, HID), jnp.float32).at[0, :2].set(b_cls)
    out = _final(xf, af, sf, cf, stats, W_msg, W_self, b_msg2, b_self2,
                 gam, bet, W_ih, W_hh, b_ih.reshape(1, 4 * HID),
                 b_hh.reshape(1, 4 * HID), wcls_p, bcls_p)
    return out[:, :2]


# speedup vs baseline: 2.8991x; 1.0093x over previous
import functools

import jax
import jax.numpy as jnp
from jax import lax
from jax.experimental import pallas as pl
from jax.experimental.pallas import tpu as pltpu
from jax.experimental.pallas import tpu_sc as plsc

N = 50000
E = 800000
T = 4
FEAT = 64
HID = 128
EDGE = 16
B = 1024
XW = T * FEAT

NCORES = 2
NSUB = 16
CHUNK = 4480
NCHUNK = 12
NPASS = NCHUNK // NCORES
NPAD = NCHUNK * CHUNK
EPT = E // NSUB
EB = 2000
NBLK = EPT // EB
SUB = 64
ROWS_PT = CHUNK // NSUB


def _edge_pass(x, src, dst, ea):
    mesh = plsc.VectorSubcoreMesh(core_axis_name="c", subcore_axis_name="s")

    @functools.partial(
        pl.kernel,
        out_type=(
            jax.ShapeDtypeStruct((NPAD, XW), jnp.float32),
            jax.ShapeDtypeStruct((NPAD, EDGE), jnp.float32),
            jax.ShapeDtypeStruct((NPAD, EDGE), jnp.float32),
        ),
        mesh=mesh,
        compiler_params=pltpu.CompilerParams(needs_layout_passes=False, use_tc_tiling_on_sc=False),
        scratch_types=dict(
            accX=pltpu.VMEM_SHARED((CHUNK + 16, XW), jnp.float32),
            accE=pltpu.VMEM_SHARED((CHUNK + 16, EDGE), jnp.float32),
            accC=pltpu.VMEM_SHARED((CHUNK + 16, EDGE), jnp.float32),
            src_blk=pltpu.VMEM((EB,), jnp.int32),
            dst_blk=pltpu.VMEM((EB,), jnp.int32),
            csrc=pltpu.VMEM((33, SUB), jnp.int32),
            cdst=pltpu.VMEM((33, SUB), jnp.int32),
            ceid=pltpu.VMEM((33, SUB), jnp.int32),
            stage_x=pltpu.VMEM((2, SUB, XW), jnp.float32),
            ea_st=pltpu.VMEM((2, SUB, EDGE), jnp.float32),
            ones_b=pltpu.VMEM((SUB, EDGE), jnp.float32),
            gsem=pltpu.SemaphoreType.DMA,
            gsem2=pltpu.SemaphoreType.DMA,
            ssem0=pltpu.SemaphoreType.DMA,
            ssem1=pltpu.SemaphoreType.DMA,
        ),
    )
    def edge_kernel(x_hbm, src_hbm, dst_hbm, ea_hbm, a_out, se_out, cn_out,
                    accX, accE, accC, src_blk, dst_blk, csrc, cdst, ceid,
                    stage_x, ea_st, ones_b, gsem, gsem2, ssem0, ssem1):
        cid = lax.axis_index("c")
        sid = lax.axis_index("s")

        zero16 = jnp.zeros((16,), jnp.float32)
        one16 = jnp.ones((16,), jnp.float32)

        def init_row(i, _):
            ones_b[i, pl.ds(0, 16)] = one16
            return 0

        lax.fori_loop(0, SUB, init_row, 0)

        def pass_body(p, _):
            lo = (NCORES * p + cid) * CHUNK
            def zrow(i, _):
                for q in range(XW // 16):
                    stage_x[0, i, pl.ds(q * 16, 16)] = zero16
                ea_st[0, i, pl.ds(0, 16)] = zero16
                return 0

            lax.fori_loop(0, SUB, zrow, 0)
            r0 = sid * ROWS_PT
            zoff = 0
            for zn in (64, 64, 64, 64, 24):
                pltpu.sync_copy(stage_x.at[0].at[pl.ds(0, zn)],
                                accX.at[pl.ds(r0 + zoff, zn)])
                pltpu.sync_copy(ea_st.at[0].at[pl.ds(0, zn)],
                                accE.at[pl.ds(r0 + zoff, zn)])
                pltpu.sync_copy(ea_st.at[0].at[pl.ds(0, zn)],
                                accC.at[pl.ds(r0 + zoff, zn)])
                zoff += zn
            @pl.when(sid == 0)
            def _():
                pltpu.sync_copy(stage_x.at[0].at[pl.ds(0, 16)],
                                accX.at[pl.ds(CHUNK, 16)])
                pltpu.sync_copy(ea_st.at[0].at[pl.ds(0, 16)],
                                accE.at[pl.ds(CHUNK, 16)])
                pltpu.sync_copy(ea_st.at[0].at[pl.ds(0, 16)],
                                accC.at[pl.ds(CHUNK, 16)])
            plsc.subcore_barrier()

            def blk_body(bk, _):
                base = sid * EPT + bk * EB
                pltpu.sync_copy(src_hbm.at[pl.ds(base, EB)], src_blk)
                pltpu.sync_copy(dst_hbm.at[pl.ds(base, EB)], dst_blk)

                def filt(i, k):
                    d = dst_blk[pl.ds(i * 16, 16)]
                    s = src_blk[pl.ds(i * 16, 16)]
                    m = (d >= lo) & (d < lo + CHUNK)
                    mi = m.astype(jnp.int32)
                    pos = k + plsc.cumsum(mi) - 1
                    pr = lax.shift_right_logical(pos, 6)
                    pc = lax.bitwise_and(pos, 63)
                    plsc.store_scatter(cdst, [pr, pc], d - lo, mask=m)
                    plsc.store_scatter(csrc, [pr, pc], s, mask=m)
                    ev = base + i * 16 + lax.iota(jnp.int32, 16)
                    plsc.store_scatter(ceid, [pr, pc], ev, mask=m)
                    return k + jnp.sum(mi)

                k = lax.fori_loop(0, EB // 16, filt, 0)
                dummy_d = jnp.full((16,), CHUNK, jnp.int32)
                zero_i = jnp.zeros((16,), jnp.int32)
                for q in range(SUB // 16):
                    pos = k + q * 16 + lax.iota(jnp.int32, 16)
                    pr = lax.shift_right_logical(pos, 6)
                    pc = lax.bitwise_and(pos, 63)
                    plsc.store_scatter(cdst, [pr, pc], dummy_d)
                    plsc.store_scatter(csrc, [pr, pc], zero_i)
                    plsc.store_scatter(ceid, [pr, pc], zero_i)
                nsub = (k + SUB - 1) // SUB

                @pl.when(nsub > 0)
                def _():
                    pltpu.async_copy(x_hbm.at[csrc.at[0]],
                                     stage_x.at[0], gsem)
                    pltpu.async_copy(ea_hbm.at[ceid.at[0]],
                                     ea_st.at[0], gsem2)

                def drain(slot_sem):
                    pltpu.make_async_copy(stage_x.at[0],
                                          accX.at[cdst.at[0]],
                                          slot_sem).wait()
                    pltpu.make_async_copy(ea_st.at[0],
                                          accE.at[cdst.at[0]],
                                          slot_sem).wait()
                    pltpu.make_async_copy(ones_b, accC.at[cdst.at[0]],
                                          slot_sem).wait()

                def sub(j, _):
                    s = j & 1
                    pltpu.make_async_copy(x_hbm.at[csrc.at[j]],
                                          stage_x.at[s], gsem).wait()
                    pltpu.make_async_copy(ea_hbm.at[ceid.at[j]],
                                          ea_st.at[s], gsem2).wait()
                    @pl.when(j + 1 < nsub)
                    def _():
                        @pl.when(j >= 1)
                        def _():
                            pl.when(s == 0)(lambda: drain(ssem1))
                            pl.when(s == 1)(lambda: drain(ssem0))
                        pltpu.async_copy(x_hbm.at[csrc.at[j + 1]],
                                         stage_x.at[1 - s], gsem)
                        pltpu.async_copy(ea_hbm.at[ceid.at[j + 1]],
                                         ea_st.at[1 - s], gsem2)
                    def fire(slot_sem):
                        pltpu.async_copy(stage_x.at[s],
                                         accX.at[cdst.at[j]],
                                         slot_sem, add=True)
                        pltpu.async_copy(ea_st.at[s],
                                         accE.at[cdst.at[j]],
                                         slot_sem, add=True)
                        pltpu.async_copy(ones_b, accC.at[cdst.at[j]],
                                         slot_sem, add=True)
                    pl.when(s == 0)(lambda: fire(ssem0))
                    pl.when(s == 1)(lambda: fire(ssem1))
                    return 0

                lax.fori_loop(0, nsub, sub, 0)
                @pl.when(nsub >= 2)
                def _():
                    pl.when((nsub & 1) == 0)(lambda: drain(ssem0))
                    pl.when((nsub & 1) == 1)(lambda: drain(ssem1))
                @pl.when(nsub >= 1)
                def _():
                    pl.when((nsub & 1) == 1)(lambda: drain(ssem0))
                    pl.when((nsub & 1) == 0)(lambda: drain(ssem1))
                return 0

            lax.fori_loop(0, NBLK, blk_body, 0)
            plsc.subcore_barrier()
            pltpu.sync_copy(accX.at[pl.ds(r0, ROWS_PT)],
                            a_out.at[pl.ds(lo + r0, ROWS_PT)])
            pltpu.sync_copy(accE.at[pl.ds(r0, ROWS_PT)],
                            se_out.at[pl.ds(lo + r0, ROWS_PT)])
            pltpu.sync_copy(accC.at[pl.ds(r0, ROWS_PT)],
                            cn_out.at[pl.ds(lo + r0, ROWS_PT)])
            return 0

        lax.fori_loop(0, NPASS, pass_body, 0)

    return edge_kernel(x, src, dst, ea)


def _focal_gather(focal, x, a, se, cn):
    mesh = plsc.VectorSubcoreMesh(core_axis_name="c", subcore_axis_name="s")
    RPW = B // (NCORES * NSUB)

    @functools.partial(
        pl.kernel,
        out_type=(
            jax.ShapeDtypeStruct((B, XW), jnp.float32),
            jax.ShapeDtypeStruct((B, XW), jnp.float32),
            jax.ShapeDtypeStruct((B, EDGE), jnp.float32),
            jax.ShapeDtypeStruct((B, EDGE), jnp.float32),
        ),
        mesh=mesh,
        compiler_params=pltpu.CompilerParams(needs_layout_passes=False, use_tc_tiling_on_sc=False),
        scratch_types=dict(
            pidx=pltpu.VMEM((RPW,), jnp.int32),
            bufx=pltpu.VMEM((RPW, XW), jnp.float32),
            bufa=pltpu.VMEM((RPW, XW), jnp.float32),
            bufs=pltpu.VMEM((RPW, EDGE), jnp.float32),
            bufc=pltpu.VMEM((RPW, EDGE), jnp.float32),
            sem=pltpu.SemaphoreType.DMA,
        ),
    )
    def focal_kernel(f_hbm, x_hbm, a_hbm, se_hbm, cn_hbm,
                     xf, af, sf, cf, pidx, bufx, bufa, bufs, bufc, sem):
        wid = lax.axis_index("s") * NCORES + lax.axis_index("c")
        base = wid * RPW
        pltpu.sync_copy(f_hbm.at[pl.ds(base, RPW)], pidx)
        pltpu.async_copy(x_hbm.at[pidx], bufx, sem).wait()
        pltpu.sync_copy(bufx, xf.at[pl.ds(base, RPW)])
        pltpu.async_copy(a_hbm.at[pidx], bufa, sem).wait()
        pltpu.sync_copy(bufa, af.at[pl.ds(base, RPW)])
        pltpu.async_copy(se_hbm.at[pidx], bufs, sem).wait()
        pltpu.sync_copy(bufs, sf.at[pl.ds(base, RPW)])
        pltpu.async_copy(cn_hbm.at[pidx], bufc, sem).wait()
        pltpu.sync_copy(bufc, cf.at[pl.ds(base, RPW)])

    return focal_kernel(focal, x, a, se, cn)


NB_STATS = 1000
NGRID = N // NB_STATS


def _stats_body(x_ref, a_ref, se_ref, cn_ref, wm_ref, ws_ref, bm_ref, bs_ref,
                o_ref):
    pc = pl.program_id(0)

    @pl.when(pc == 0)
    def _():
        o_ref[...] = jnp.zeros_like(o_ref)

    wx = wm_ref[0:FEAT, :]
    we = wm_ref[FEAT:FEAT + EDGE, :]
    ws = ws_ref[...]
    cn = cn_ref[:, 0:1]
    basem = jnp.dot(se_ref[...], we, preferred_element_type=jnp.float32)
    basem = basem + cn * bm_ref[...]
    rinv = 1.0 / jnp.maximum(cn, 1.0)
    rows = []
    rows2 = []
    for t in range(T):
        xt = x_ref[:, FEAT * t:FEAT * (t + 1)]
        at = a_ref[:, FEAT * t:FEAT * (t + 1)]
        out = jnp.dot(xt, ws, preferred_element_type=jnp.float32)
        out = out + bs_ref[...]
        out = out + (jnp.dot(at, wx, preferred_element_type=jnp.float32)
                     + basem) * rinv
        rows.append(jnp.sum(out, axis=0, keepdims=True))
        rows2.append(jnp.sum(out * out, axis=0, keepdims=True))
    o_ref[...] += jnp.concatenate(rows + rows2, axis=0)


def _stats(x, a, se, cn, w_msg, w_self, b_msg, b_self):
    return pl.pallas_call(
        _stats_body,
        grid=(NGRID,),
        in_specs=[
            pl.BlockSpec((NB_STATS, XW), lambda i: (i, 0)),
            pl.BlockSpec((NB_STATS, XW), lambda i: (i, 0)),
            pl.BlockSpec((NB_STATS, EDGE), lambda i: (i, 0)),
            pl.BlockSpec((NB_STATS, EDGE), lambda i: (i, 0)),
            pl.BlockSpec((FEAT + EDGE, HID), lambda i: (0, 0)),
            pl.BlockSpec((FEAT, HID), lambda i: (0, 0)),
            pl.BlockSpec((1, HID), lambda i: (0, 0)),
            pl.BlockSpec((1, HID), lambda i: (0, 0)),
        ],
        out_specs=pl.BlockSpec((2 * T, HID), lambda i: (0, 0)),
        out_shape=jax.ShapeDtypeStruct((2 * T, HID), jnp.float32),
    )(x, a, se, cn, w_msg, w_self, b_msg, b_self)


def _final_body(xf_ref, af_ref, sf_ref, cf_ref, st_ref, wm_ref, ws_ref,
                bm_ref, bs_ref, g_ref, bb_ref, wih_ref, whh_ref, bih_ref,
                bhh_ref, wcls_ref, bcls_ref, o_ref):
    wx = wm_ref[0:FEAT, :]
    we = wm_ref[FEAT:FEAT + EDGE, :]
    ws = ws_ref[...]
    mean = st_ref[0:T, :] * (1.0 / N)
    var = st_ref[T:2 * T, :] * (1.0 / N) - mean * mean
    scale = g_ref[...] * lax.rsqrt(var + 1e-5)
    shift = bb_ref[...] - mean * scale
    cn = cf_ref[:, 0:1]
    basem = jnp.dot(sf_ref[...], we, preferred_element_type=jnp.float32)
    basem = basem + cn * bm_ref[...]
    rinv = 1.0 / jnp.maximum(cn, 1.0)
    h = jnp.zeros((B, HID), jnp.float32)
    c = jnp.zeros((B, HID), jnp.float32)
    for t in range(T):
        xt = xf_ref[:, FEAT * t:FEAT * (t + 1)]
        at = af_ref[:, FEAT * t:FEAT * (t + 1)]
        out = jnp.dot(xt, ws, preferred_element_type=jnp.float32)
        out = out + bs_ref[...]
        out = out + (jnp.dot(at, wx, preferred_element_type=jnp.float32)
                     + basem) * rinv
        ht = jnp.maximum(out * scale[t:t + 1, :] + shift[t:t + 1, :], 0.0)
        gates = (jnp.dot(ht, wih_ref[...], preferred_element_type=jnp.float32)
                 + bih_ref[...]
                 + jnp.dot(h, whh_ref[...], preferred_element_type=jnp.float32)
                 + bhh_ref[...])
        i_g = jax.nn.sigmoid(gates[:, 0 * HID:1 * HID])
        f_g = jax.nn.sigmoid(gates[:, 1 * HID:2 * HID])
        g_g = jnp.tanh(gates[:, 2 * HID:3 * HID])
        o_g = jax.nn.sigmoid(gates[:, 3 * HID:4 * HID])
        c = f_g * c + i_g * g_g
        h = o_g * jnp.tanh(c)
    o_ref[...] = jnp.dot(h, wcls_ref[...],
                         preferred_element_type=jnp.float32) + bcls_ref[...]


def _final(xf, af, sf, cf, stats, w_msg, w_self, b_msg, b_self, gam, bet,
           w_ih, w_hh, b_ih, b_hh, wcls_p, bcls_p):
    return pl.pallas_call(
        _final_body,
        out_shape=jax.ShapeDtypeStruct((B, HID), jnp.float32),
    )(xf, af, sf, cf, stats, w_msg, w_self, b_msg, b_self, gam, bet,
      w_ih, w_hh, b_ih, b_hh, wcls_p, bcls_p)


def kernel(x, edge_index, edge_attr, ptr, W_msg, b_msg, W_self, b_self,
           bn_gamma, bn_beta, W_ih, W_hh, b_ih, b_hh, W_cls, b_cls):
    src = edge_index[0]
    dst = edge_index[1]
    focal = ptr[:B]

    a, se, cn = _edge_pass(x, src, dst, edge_attr)
    xf, af, sf, cf = _focal_gather(focal, x, a, se, cn)

    b_msg2 = b_msg.reshape(1, HID)
    b_self2 = b_self.reshape(1, HID)
    stats = _stats(x, a, se, cn, W_msg, W_self, b_msg2, b_self2)

    gam = jnp.broadcast_to(bn_gamma.reshape(1, HID), (T, HID))
    bet = jnp.broadcast_to(bn_beta.reshape(1, HID), (T, HID))
    wcls_p = jnp.zeros((HID, HID), jnp.float32).at[:, :2].set(W_cls)
    bcls_p = jnp.zeros((1, HID), jnp.float32).at[0, :2].set(b_cls)
    out = _final(xf, af, sf, cf, stats, W_msg, W_self, b_msg2, b_self2,
                 gam, bet, W_ih, W_hh, b_ih.reshape(1, 4 * HID),
                 b_hh.reshape(1, 4 * HID), wcls_p, bcls_p)
    return out[:, :2]


# speedup vs baseline: 4.0933x; 1.4119x over previous
import functools

import jax
import jax.numpy as jnp
from jax import lax
from jax.experimental import pallas as pl
from jax.experimental.pallas import tpu as pltpu
from jax.experimental.pallas import tpu_sc as plsc

N = 50000
E = 800000
T = 4
FEAT = 64
HID = 128
EDGE = 16
B = 1024
XW = T * FEAT

NCORES = 2
NSUB = 16
CHUNK = 4480
NCHUNK = 12
NPASS = NCHUNK // NCORES
NPAD = NCHUNK * CHUNK
EPT = E // NSUB
EB = 2000
NBLK = EPT // EB
SUB = 32
NSLOT = 4
ROWS_PT = CHUNK // NSUB


def _edge_pass(x, src, dst, ea):
    mesh = plsc.VectorSubcoreMesh(core_axis_name="c", subcore_axis_name="s")

    @functools.partial(
        pl.kernel,
        out_type=(
            jax.ShapeDtypeStruct((NPAD, XW), jnp.float32),
            jax.ShapeDtypeStruct((NPAD, EDGE), jnp.float32),
            jax.ShapeDtypeStruct((NPAD, EDGE), jnp.float32),
        ),
        mesh=mesh,
        compiler_params=pltpu.CompilerParams(needs_layout_passes=False, use_tc_tiling_on_sc=False),
        scratch_types=dict(
            accX=pltpu.VMEM_SHARED((CHUNK + 16, XW), jnp.float32),
            accE=pltpu.VMEM_SHARED((CHUNK + 16, EDGE), jnp.float32),
            accC=pltpu.VMEM_SHARED((CHUNK + 16, EDGE), jnp.float32),
            src_blk=pltpu.VMEM((EB,), jnp.int32),
            dst_blk=pltpu.VMEM((EB,), jnp.int32),
            csrc=pltpu.VMEM((65, SUB), jnp.int32),
            cdst=pltpu.VMEM((65, SUB), jnp.int32),
            ceid=pltpu.VMEM((65, SUB), jnp.int32),
            stage_x=pltpu.VMEM((NSLOT, SUB, XW), jnp.float32),
            ea_st=pltpu.VMEM((NSLOT, SUB, EDGE), jnp.float32),
            ones_b=pltpu.VMEM((SUB, EDGE), jnp.float32),
            gxs=pltpu.SemaphoreType.DMA((NSLOT,)),
            ges=pltpu.SemaphoreType.DMA((NSLOT,)),
            scs=pltpu.SemaphoreType.DMA((NSLOT,)),
        ),
    )
    def edge_kernel(x_hbm, src_hbm, dst_hbm, ea_hbm, a_out, se_out, cn_out,
                    accX, accE, accC, src_blk, dst_blk, csrc, cdst, ceid,
                    stage_x, ea_st, ones_b, gxs, ges, scs):
        cid = lax.axis_index("c")
        sid = lax.axis_index("s")

        zero16 = jnp.zeros((16,), jnp.float32)
        one16 = jnp.ones((16,), jnp.float32)

        def init_row(i, _):
            ones_b[i, pl.ds(0, 16)] = one16
            return 0

        lax.fori_loop(0, SUB, init_row, 0)

        def pass_body(p, _):
            lo = (NCORES * p + cid) * CHUNK
            def zrow(i, _):
                for q in range(XW // 16):
                    stage_x[0, i, pl.ds(q * 16, 16)] = zero16
                ea_st[0, i, pl.ds(0, 16)] = zero16
                return 0

            lax.fori_loop(0, SUB, zrow, 0)
            r0 = sid * ROWS_PT
            zoff = 0
            for zn in (32, 32, 32, 32, 32, 32, 32, 32, 24):
                pltpu.sync_copy(stage_x.at[0].at[pl.ds(0, zn)],
                                accX.at[pl.ds(r0 + zoff, zn)])
                pltpu.sync_copy(ea_st.at[0].at[pl.ds(0, zn)],
                                accE.at[pl.ds(r0 + zoff, zn)])
                pltpu.sync_copy(ea_st.at[0].at[pl.ds(0, zn)],
                                accC.at[pl.ds(r0 + zoff, zn)])
                zoff += zn
            @pl.when(sid == 0)
            def _():
                pltpu.sync_copy(stage_x.at[0].at[pl.ds(0, 16)],
                                accX.at[pl.ds(CHUNK, 16)])
                pltpu.sync_copy(ea_st.at[0].at[pl.ds(0, 16)],
                                accE.at[pl.ds(CHUNK, 16)])
                pltpu.sync_copy(ea_st.at[0].at[pl.ds(0, 16)],
                                accC.at[pl.ds(CHUNK, 16)])
            plsc.subcore_barrier()

            def blk_body(bk, _):
                base = sid * EPT + bk * EB
                pltpu.sync_copy(src_hbm.at[pl.ds(base, EB)], src_blk)
                pltpu.sync_copy(dst_hbm.at[pl.ds(base, EB)], dst_blk)

                def filt(i, k):
                    d = dst_blk[pl.ds(i * 16, 16)]
                    s = src_blk[pl.ds(i * 16, 16)]
                    m = (d >= lo) & (d < lo + CHUNK)
                    mi = m.astype(jnp.int32)
                    pos = k + plsc.cumsum(mi) - 1
                    pr = lax.shift_right_logical(pos, 5)
                    pc = lax.bitwise_and(pos, 31)
                    plsc.store_scatter(cdst, [pr, pc], d - lo, mask=m)
                    plsc.store_scatter(csrc, [pr, pc], s, mask=m)
                    ev = base + i * 16 + lax.iota(jnp.int32, 16)
                    plsc.store_scatter(ceid, [pr, pc], ev, mask=m)
                    return k + jnp.sum(mi)

                k = lax.fori_loop(0, EB // 16, filt, 0)
                dummy_d = jnp.full((16,), CHUNK, jnp.int32)
                zero_i = jnp.zeros((16,), jnp.int32)
                for q in range(SUB // 16):
                    pos = k + q * 16 + lax.iota(jnp.int32, 16)
                    pr = lax.shift_right_logical(pos, 5)
                    pc = lax.bitwise_and(pos, 31)
                    plsc.store_scatter(cdst, [pr, pc], dummy_d)
                    plsc.store_scatter(csrc, [pr, pc], zero_i)
                    plsc.store_scatter(ceid, [pr, pc], zero_i)
                nsub = (k + SUB - 1) // SUB

                def issue(jj, sl):
                    pltpu.async_copy(x_hbm.at[csrc.at[jj]],
                                     stage_x.at[sl], gxs.at[sl])
                    pltpu.async_copy(ea_hbm.at[ceid.at[jj]],
                                     ea_st.at[sl], ges.at[sl])

                def drain(sl):
                    pltpu.make_async_copy(stage_x.at[0],
                                          accX.at[cdst.at[0]],
                                          scs.at[sl]).wait()
                    pltpu.make_async_copy(ea_st.at[0],
                                          accE.at[cdst.at[0]],
                                          scs.at[sl]).wait()
                    pltpu.make_async_copy(ones_b, accC.at[cdst.at[0]],
                                          scs.at[sl]).wait()

                for jj in range(NSLOT - 1):
                    @pl.when(jj < nsub)
                    def _(jj=jj):
                        issue(jj, jj)

                def sub(j, _):
                    s = j & 3
                    pltpu.make_async_copy(x_hbm.at[csrc.at[j]],
                                          stage_x.at[s], gxs.at[s]).wait()
                    pltpu.make_async_copy(ea_hbm.at[ceid.at[j]],
                                          ea_st.at[s], ges.at[s]).wait()
                    @pl.when(j + NSLOT - 1 < nsub)
                    def _():
                        @pl.when(j >= 1)
                        def _():
                            drain((s + 3) & 3)
                        issue(j + NSLOT - 1, (s + 3) & 3)
                    pltpu.async_copy(stage_x.at[s], accX.at[cdst.at[j]],
                                     scs.at[s], add=True)
                    pltpu.async_copy(ea_st.at[s], accE.at[cdst.at[j]],
                                     scs.at[s], add=True)
                    pltpu.async_copy(ones_b, accC.at[cdst.at[j]],
                                     scs.at[s], add=True)
                    return 0

                lax.fori_loop(0, nsub, sub, 0)
                for ss in range(NSLOT):
                    @pl.when(ss < nsub)
                    def _(ss=ss):
                        drain(ss)
                return 0

            lax.fori_loop(0, NBLK, blk_body, 0)
            plsc.subcore_barrier()
            pltpu.sync_copy(accX.at[pl.ds(r0, ROWS_PT)],
                            a_out.at[pl.ds(lo + r0, ROWS_PT)])
            pltpu.sync_copy(accE.at[pl.ds(r0, ROWS_PT)],
                            se_out.at[pl.ds(lo + r0, ROWS_PT)])
            pltpu.sync_copy(accC.at[pl.ds(r0, ROWS_PT)],
                            cn_out.at[pl.ds(lo + r0, ROWS_PT)])
            return 0

        lax.fori_loop(0, NPASS, pass_body, 0)

    return edge_kernel(x, src, dst, ea)


def _focal_gather(focal, x, a, se, cn):
    mesh = plsc.VectorSubcoreMesh(core_axis_name="c", subcore_axis_name="s")
    RPW = B // (NCORES * NSUB)

    @functools.partial(
        pl.kernel,
        out_type=(
            jax.ShapeDtypeStruct((B, XW), jnp.float32),
            jax.ShapeDtypeStruct((B, XW), jnp.float32),
            jax.ShapeDtypeStruct((B, EDGE), jnp.float32),
            jax.ShapeDtypeStruct((B, EDGE), jnp.float32),
        ),
        mesh=mesh,
        compiler_params=pltpu.CompilerParams(needs_layout_passes=False, use_tc_tiling_on_sc=False),
        scratch_types=dict(
            pidx=pltpu.VMEM((RPW,), jnp.int32),
            bufx=pltpu.VMEM((RPW, XW), jnp.float32),
            bufa=pltpu.VMEM((RPW, XW), jnp.float32),
            bufs=pltpu.VMEM((RPW, EDGE), jnp.float32),
            bufc=pltpu.VMEM((RPW, EDGE), jnp.float32),
            sem=pltpu.SemaphoreType.DMA,
        ),
    )
    def focal_kernel(f_hbm, x_hbm, a_hbm, se_hbm, cn_hbm,
                     xf, af, sf, cf, pidx, bufx, bufa, bufs, bufc, sem):
        wid = lax.axis_index("s") * NCORES + lax.axis_index("c")
        base = wid * RPW
        pltpu.sync_copy(f_hbm.at[pl.ds(base, RPW)], pidx)
        pltpu.async_copy(x_hbm.at[pidx], bufx, sem).wait()
        pltpu.sync_copy(bufx, xf.at[pl.ds(base, RPW)])
        pltpu.async_copy(a_hbm.at[pidx], bufa, sem).wait()
        pltpu.sync_copy(bufa, af.at[pl.ds(base, RPW)])
        pltpu.async_copy(se_hbm.at[pidx], bufs, sem).wait()
        pltpu.sync_copy(bufs, sf.at[pl.ds(base, RPW)])
        pltpu.async_copy(cn_hbm.at[pidx], bufc, sem).wait()
        pltpu.sync_copy(bufc, cf.at[pl.ds(base, RPW)])

    return focal_kernel(focal, x, a, se, cn)


NB_STATS = 1000
NGRID = N // NB_STATS


def _stats_body(x_ref, a_ref, se_ref, cn_ref, wm_ref, ws_ref, bm_ref, bs_ref,
                o_ref):
    pc = pl.program_id(0)

    @pl.when(pc == 0)
    def _():
        o_ref[...] = jnp.zeros_like(o_ref)

    wx = wm_ref[0:FEAT, :]
    we = wm_ref[FEAT:FEAT + EDGE, :]
    ws = ws_ref[...]
    cn = cn_ref[:, 0:1]
    basem = jnp.dot(se_ref[...], we, preferred_element_type=jnp.float32)
    basem = basem + cn * bm_ref[...]
    rinv = 1.0 / jnp.maximum(cn, 1.0)
    rows = []
    rows2 = []
    for t in range(T):
        xt = x_ref[:, FEAT * t:FEAT * (t + 1)]
        at = a_ref[:, FEAT * t:FEAT * (t + 1)]
        out = jnp.dot(xt, ws, preferred_element_type=jnp.float32)
        out = out + bs_ref[...]
        out = out + (jnp.dot(at, wx, preferred_element_type=jnp.float32)
                     + basem) * rinv
        rows.append(jnp.sum(out, axis=0, keepdims=True))
        rows2.append(jnp.sum(out * out, axis=0, keepdims=True))
    o_ref[...] += jnp.concatenate(rows + rows2, axis=0)


def _stats(x, a, se, cn, w_msg, w_self, b_msg, b_self):
    return pl.pallas_call(
        _stats_body,
        grid=(NGRID,),
        in_specs=[
            pl.BlockSpec((NB_STATS, XW), lambda i: (i, 0)),
            pl.BlockSpec((NB_STATS, XW), lambda i: (i, 0)),
            pl.BlockSpec((NB_STATS, EDGE), lambda i: (i, 0)),
            pl.BlockSpec((NB_STATS, EDGE), lambda i: (i, 0)),
            pl.BlockSpec((FEAT + EDGE, HID), lambda i: (0, 0)),
            pl.BlockSpec((FEAT, HID), lambda i: (0, 0)),
            pl.BlockSpec((1, HID), lambda i: (0, 0)),
            pl.BlockSpec((1, HID), lambda i: (0, 0)),
        ],
        out_specs=pl.BlockSpec((2 * T, HID), lambda i: (0, 0)),
        out_shape=jax.ShapeDtypeStruct((2 * T, HID), jnp.float32),
    )(x, a, se, cn, w_msg, w_self, b_msg, b_self)


def _final_body(xf_ref, af_ref, sf_ref, cf_ref, st_ref, wm_ref, ws_ref,
                bm_ref, bs_ref, g_ref, bb_ref, wih_ref, whh_ref, bih_ref,
                bhh_ref, wcls_ref, bcls_ref, o_ref):
    wx = wm_ref[0:FEAT, :]
    we = wm_ref[FEAT:FEAT + EDGE, :]
    ws = ws_ref[...]
    mean = st_ref[0:T, :] * (1.0 / N)
    var = st_ref[T:2 * T, :] * (1.0 / N) - mean * mean
    scale = g_ref[...] * lax.rsqrt(var + 1e-5)
    shift = bb_ref[...] - mean * scale
    cn = cf_ref[:, 0:1]
    basem = jnp.dot(sf_ref[...], we, preferred_element_type=jnp.float32)
    basem = basem + cn * bm_ref[...]
    rinv = 1.0 / jnp.maximum(cn, 1.0)
    h = jnp.zeros((B, HID), jnp.float32)
    c = jnp.zeros((B, HID), jnp.float32)
    for t in range(T):
        xt = xf_ref[:, FEAT * t:FEAT * (t + 1)]
        at = af_ref[:, FEAT * t:FEAT * (t + 1)]
        out = jnp.dot(xt, ws, preferred_element_type=jnp.float32)
        out = out + bs_ref[...]
        out = out + (jnp.dot(at, wx, preferred_element_type=jnp.float32)
                     + basem) * rinv
        ht = jnp.maximum(out * scale[t:t + 1, :] + shift[t:t + 1, :], 0.0)
        gates = (jnp.dot(ht, wih_ref[...], preferred_element_type=jnp.float32)
                 + bih_ref[...]
                 + jnp.dot(h, whh_ref[...], preferred_element_type=jnp.float32)
                 + bhh_ref[...])
        i_g = jax.nn.sigmoid(gates[:, 0 * HID:1 * HID])
        f_g = jax.nn.sigmoid(gates[:, 1 * HID:2 * HID])
        g_g = jnp.tanh(gates[:, 2 * HID:3 * HID])
        o_g = jax.nn.sigmoid(gates[:, 3 * HID:4 * HID])
        c = f_g * c + i_g * g_g
        h = o_g * jnp.tanh(c)
    o_ref[...] = jnp.dot(h, wcls_ref[...],
                         preferred_element_type=jnp.float32) + bcls_ref[...]


def _final(xf, af, sf, cf, stats, w_msg, w_self, b_msg, b_self, gam, bet,
           w_ih, w_hh, b_ih, b_hh, wcls_p, bcls_p):
    return pl.pallas_call(
        _final_body,
        out_shape=jax.ShapeDtypeStruct((B, HID), jnp.float32),
    )(xf, af, sf, cf, stats, w_msg, w_self, b_msg, b_self, gam, bet,
      w_ih, w_hh, b_ih, b_hh, wcls_p, bcls_p)


def kernel(x, edge_index, edge_attr, ptr, W_msg, b_msg, W_self, b_self,
           bn_gamma, bn_beta, W_ih, W_hh, b_ih, b_hh, W_cls, b_cls):
    src = edge_index[0]
    dst = edge_index[1]
    focal = ptr[:B]

    a, se, cn = _edge_pass(x, src, dst, edge_attr)
    xf, af, sf, cf = _focal_gather(focal, x, a, se, cn)

    b_msg2 = b_msg.reshape(1, HID)
    b_self2 = b_self.reshape(1, HID)
    stats = _stats(x, a, se, cn, W_msg, W_self, b_msg2, b_self2)

    gam = jnp.broadcast_to(bn_gamma.reshape(1, HID), (T, HID))
    bet = jnp.broadcast_to(bn_beta.reshape(1, HID), (T, HID))
    wcls_p = jnp.zeros((HID, HID), jnp.float32).at[:, :2].set(W_cls)
    bcls_p = jnp.zeros((1, HID), jnp.float32).at[0, :2].set(b_cls)
    out = _final(xf, af, sf, cf, stats, W_msg, W_self, b_msg2, b_self2,
                 gam, bet, W_ih, W_hh, b_ih.reshape(1, 4 * HID),
                 b_hh.reshape(1, 4 * HID), wcls_p, bcls_p)
    return out[:, :2]


# speedup vs baseline: 8.9912x; 2.1966x over previous
import functools

import jax
import jax.numpy as jnp
from jax import lax
from jax.experimental import pallas as pl
from jax.experimental.pallas import tpu as pltpu
from jax.experimental.pallas import tpu_sc as plsc

N = 50000
E = 800000
T = 4
FEAT = 64
HID = 128
EDGE = 16
B = 1024
XW = T * FEAT

NCORES = 2
NSUB = 16
CHUNK = 4480
NCHUNK = 12
NPASS = NCHUNK // NCORES
NPAD = NCHUNK * CHUNK
EPT = E // NSUB
EB = 2000
NBLK = EPT // EB
SUB = 32
NSLOT = 4
ROWS_PT = CHUNK // NSUB


def _edge_pass(x, src, dst, ea):
    mesh = plsc.VectorSubcoreMesh(core_axis_name="c", subcore_axis_name="s")

    @functools.partial(
        pl.kernel,
        out_type=(
            jax.ShapeDtypeStruct((NPAD, XW), jnp.float32),
            jax.ShapeDtypeStruct((NPAD, EDGE), jnp.float32),
            jax.ShapeDtypeStruct((NPAD, EDGE), jnp.float32),
        ),
        mesh=mesh,
        compiler_params=pltpu.CompilerParams(needs_layout_passes=False, use_tc_tiling_on_sc=False),
        scratch_types=dict(
            accX=pltpu.VMEM_SHARED((CHUNK + 16, XW), jnp.float32),
            accE=pltpu.VMEM_SHARED((CHUNK + 16, EDGE), jnp.float32),
            accC=pltpu.VMEM_SHARED((CHUNK + 16, EDGE), jnp.float32),
            src_blk=pltpu.VMEM((EB,), jnp.int32),
            dst_blk=pltpu.VMEM((EB,), jnp.int32),
            csrc=pltpu.VMEM((65, SUB), jnp.int32),
            cdst=pltpu.VMEM((65, SUB), jnp.int32),
            ceid=pltpu.VMEM((65, SUB), jnp.int32),
            stage_x=pltpu.VMEM((NSLOT, SUB, XW), jnp.float32),
            ea_st=pltpu.VMEM((NSLOT, SUB, EDGE), jnp.float32),
            ones_b=pltpu.VMEM((SUB, EDGE), jnp.float32),
            gxs=pltpu.SemaphoreType.DMA((NSLOT,)),
            ges=pltpu.SemaphoreType.DMA((NSLOT,)),
            scs=pltpu.SemaphoreType.DMA((NSLOT,)),
        ),
    )
    def edge_kernel(x_hbm, src_hbm, dst_hbm, ea_hbm, a_out, se_out, cn_out,
                    accX, accE, accC, src_blk, dst_blk, csrc, cdst, ceid,
                    stage_x, ea_st, ones_b, gxs, ges, scs):
        cid = lax.axis_index("c")
        sid = lax.axis_index("s")

        zero16 = jnp.zeros((16,), jnp.float32)
        one16 = jnp.ones((16,), jnp.float32)

        def init_row(i, _):
            ones_b[i, pl.ds(0, 16)] = one16
            return 0

        lax.fori_loop(0, SUB, init_row, 0)

        def pass_body(p, _):
            lo = (NCORES * p + cid) * CHUNK
            def zrow(i, _):
                for q in range(XW // 16):
                    stage_x[0, i, pl.ds(q * 16, 16)] = zero16
                ea_st[0, i, pl.ds(0, 16)] = zero16
                return 0

            lax.fori_loop(0, SUB, zrow, 0)
            r0 = sid * ROWS_PT
            zoff = 0
            for zn in (32, 32, 32, 32, 32, 32, 32, 32, 24):
                pltpu.sync_copy(stage_x.at[0].at[pl.ds(0, zn)],
                                accX.at[pl.ds(r0 + zoff, zn)])
                pltpu.sync_copy(ea_st.at[0].at[pl.ds(0, zn)],
                                accE.at[pl.ds(r0 + zoff, zn)])
                pltpu.sync_copy(ea_st.at[0].at[pl.ds(0, zn)],
                                accC.at[pl.ds(r0 + zoff, zn)])
                zoff += zn
            @pl.when(sid == 0)
            def _():
                pltpu.sync_copy(stage_x.at[0].at[pl.ds(0, 16)],
                                accX.at[pl.ds(CHUNK, 16)])
                pltpu.sync_copy(ea_st.at[0].at[pl.ds(0, 16)],
                                accE.at[pl.ds(CHUNK, 16)])
                pltpu.sync_copy(ea_st.at[0].at[pl.ds(0, 16)],
                                accC.at[pl.ds(CHUNK, 16)])
            plsc.subcore_barrier()

            def blk_body(bk, kin):
                base = sid * EPT + bk * EB
                pltpu.sync_copy(src_hbm.at[pl.ds(base, EB)], src_blk)
                pltpu.sync_copy(dst_hbm.at[pl.ds(base, EB)], dst_blk)

                def filt(i, k):
                    d = dst_blk[pl.ds(i * 16, 16)]
                    s = src_blk[pl.ds(i * 16, 16)]
                    m = (d >= lo) & (d < lo + CHUNK)
                    mi = m.astype(jnp.int32)
                    pos = k + plsc.cumsum(mi) - 1
                    pr = lax.shift_right_logical(pos, 5)
                    pc = lax.bitwise_and(pos, 31)
                    plsc.store_scatter(cdst, [pr, pc], d - lo, mask=m)
                    plsc.store_scatter(csrc, [pr, pc], s, mask=m)
                    ev = base + i * 16 + lax.iota(jnp.int32, 16)
                    plsc.store_scatter(ceid, [pr, pc], ev, mask=m)
                    return k + jnp.sum(mi)

                k = lax.fori_loop(0, EB // 16, filt, kin, unroll=4)
                nsub = k // SUB

                def issue(jj, sl):
                    pltpu.async_copy(x_hbm.at[csrc.at[jj]],
                                     stage_x.at[sl], gxs.at[sl])
                    pltpu.async_copy(ea_hbm.at[ceid.at[jj]],
                                     ea_st.at[sl], ges.at[sl])

                def drain(sl):
                    pltpu.make_async_copy(stage_x.at[0],
                                          accX.at[cdst.at[0]],
                                          scs.at[sl]).wait()
                    pltpu.make_async_copy(ea_st.at[0],
                                          accE.at[cdst.at[0]],
                                          scs.at[sl]).wait()
                    pltpu.make_async_copy(ones_b, accC.at[cdst.at[0]],
                                          scs.at[sl]).wait()

                for jj in range(NSLOT - 1):
                    @pl.when(jj < nsub)
                    def _(jj=jj):
                        issue(jj, jj)

                def sub(j, _):
                    s = j & 3
                    pltpu.make_async_copy(x_hbm.at[csrc.at[j]],
                                          stage_x.at[s], gxs.at[s]).wait()
                    pltpu.make_async_copy(ea_hbm.at[ceid.at[j]],
                                          ea_st.at[s], ges.at[s]).wait()
                    @pl.when(j + NSLOT - 1 < nsub)
                    def _():
                        @pl.when(j >= 1)
                        def _():
                            drain((s + 3) & 3)
                        issue(j + NSLOT - 1, (s + 3) & 3)
                    pltpu.async_copy(stage_x.at[s], accX.at[cdst.at[j]],
                                     scs.at[s], add=True)
                    pltpu.async_copy(ea_st.at[s], accE.at[cdst.at[j]],
                                     scs.at[s], add=True)
                    pltpu.async_copy(ones_b, accC.at[cdst.at[j]],
                                     scs.at[s], add=True)
                    return 0

                lax.fori_loop(0, nsub, sub, 0)
                for ss in range(NSLOT):
                    @pl.when(ss < nsub)
                    def _(ss=ss):
                        drain(ss)
                for q in range(SUB // 16):
                    cdst[0, pl.ds(q * 16, 16)] = cdst[nsub, pl.ds(q * 16, 16)]
                    csrc[0, pl.ds(q * 16, 16)] = csrc[nsub, pl.ds(q * 16, 16)]
                    ceid[0, pl.ds(q * 16, 16)] = ceid[nsub, pl.ds(q * 16, 16)]
                return k & (SUB - 1)

            kfl = lax.fori_loop(0, NBLK, blk_body, 0)
            @pl.when(kfl > 0)
            def _():
                dummy_d = jnp.full((16,), CHUNK, jnp.int32)
                zero_i = jnp.zeros((16,), jnp.int32)
                for q in range(SUB // 16):
                    pos = kfl + q * 16 + lax.iota(jnp.int32, 16)
                    pr = lax.shift_right_logical(pos, 5)
                    pc = lax.bitwise_and(pos, 31)
                    plsc.store_scatter(cdst, [pr, pc], dummy_d)
                    plsc.store_scatter(csrc, [pr, pc], zero_i)
                    plsc.store_scatter(ceid, [pr, pc], zero_i)
                pltpu.async_copy(x_hbm.at[csrc.at[0]], stage_x.at[0],
                                 gxs.at[0])
                pltpu.async_copy(ea_hbm.at[ceid.at[0]], ea_st.at[0],
                                 ges.at[0])
                pltpu.make_async_copy(x_hbm.at[csrc.at[0]], stage_x.at[0],
                                      gxs.at[0]).wait()
                pltpu.make_async_copy(ea_hbm.at[ceid.at[0]], ea_st.at[0],
                                      ges.at[0]).wait()
                pltpu.sync_copy(stage_x.at[0], accX.at[cdst.at[0]],
                                add=True)
                pltpu.sync_copy(ea_st.at[0], accE.at[cdst.at[0]],
                                add=True)
                pltpu.sync_copy(ones_b, accC.at[cdst.at[0]], add=True)
            plsc.subcore_barrier()
            pltpu.sync_copy(accX.at[pl.ds(r0, ROWS_PT)],
                            a_out.at[pl.ds(lo + r0, ROWS_PT)])
            pltpu.sync_copy(accE.at[pl.ds(r0, ROWS_PT)],
                            se_out.at[pl.ds(lo + r0, ROWS_PT)])
            pltpu.sync_copy(accC.at[pl.ds(r0, ROWS_PT)],
                            cn_out.at[pl.ds(lo + r0, ROWS_PT)])
            return 0

        lax.fori_loop(0, NPASS, pass_body, 0)

    return edge_kernel(x, src, dst, ea)


def _focal_gather(focal, x, a, se, cn):
    mesh = plsc.VectorSubcoreMesh(core_axis_name="c", subcore_axis_name="s")
    RPW = B // (NCORES * NSUB)

    @functools.partial(
        pl.kernel,
        out_type=(
            jax.ShapeDtypeStruct((B, XW), jnp.float32),
            jax.ShapeDtypeStruct((B, XW), jnp.float32),
            jax.ShapeDtypeStruct((B, EDGE), jnp.float32),
            jax.ShapeDtypeStruct((B, EDGE), jnp.float32),
        ),
        mesh=mesh,
        compiler_params=pltpu.CompilerParams(needs_layout_passes=False, use_tc_tiling_on_sc=False),
        scratch_types=dict(
            pidx=pltpu.VMEM((RPW,), jnp.int32),
            bufx=pltpu.VMEM((RPW, XW), jnp.float32),
            bufa=pltpu.VMEM((RPW, XW), jnp.float32),
            bufs=pltpu.VMEM((RPW, EDGE), jnp.float32),
            bufc=pltpu.VMEM((RPW, EDGE), jnp.float32),
            sem=pltpu.SemaphoreType.DMA,
        ),
    )
    def focal_kernel(f_hbm, x_hbm, a_hbm, se_hbm, cn_hbm,
                     xf, af, sf, cf, pidx, bufx, bufa, bufs, bufc, sem):
        wid = lax.axis_index("s") * NCORES + lax.axis_index("c")
        base = wid * RPW
        pltpu.sync_copy(f_hbm.at[pl.ds(base, RPW)], pidx)
        pltpu.async_copy(x_hbm.at[pidx], bufx, sem).wait()
        pltpu.sync_copy(bufx, xf.at[pl.ds(base, RPW)])
        pltpu.async_copy(a_hbm.at[pidx], bufa, sem).wait()
        pltpu.sync_copy(bufa, af.at[pl.ds(base, RPW)])
        pltpu.async_copy(se_hbm.at[pidx], bufs, sem).wait()
        pltpu.sync_copy(bufs, sf.at[pl.ds(base, RPW)])
        pltpu.async_copy(cn_hbm.at[pidx], bufc, sem).wait()
        pltpu.sync_copy(bufc, cf.at[pl.ds(base, RPW)])

    return focal_kernel(focal, x, a, se, cn)


NB_STATS = 1000
NGRID = N // NB_STATS


def _stats_body(x_ref, a_ref, se_ref, cn_ref, wm_ref, ws_ref, bm_ref, bs_ref,
                o_ref):
    pc = pl.program_id(0)

    @pl.when(pc == 0)
    def _():
        o_ref[...] = jnp.zeros_like(o_ref)

    wx = wm_ref[0:FEAT, :]
    we = wm_ref[FEAT:FEAT + EDGE, :]
    ws = ws_ref[...]
    cn = cn_ref[:, 0:1]
    basem = jnp.dot(se_ref[...], we, preferred_element_type=jnp.float32)
    basem = basem + cn * bm_ref[...]
    rinv = 1.0 / jnp.maximum(cn, 1.0)
    rows = []
    rows2 = []
    for t in range(T):
        xt = x_ref[:, FEAT * t:FEAT * (t + 1)]
        at = a_ref[:, FEAT * t:FEAT * (t + 1)]
        out = jnp.dot(xt, ws, preferred_element_type=jnp.float32)
        out = out + bs_ref[...]
        out = out + (jnp.dot(at, wx, preferred_element_type=jnp.float32)
                     + basem) * rinv
        rows.append(jnp.sum(out, axis=0, keepdims=True))
        rows2.append(jnp.sum(out * out, axis=0, keepdims=True))
    o_ref[...] += jnp.concatenate(rows + rows2, axis=0)


def _stats(x, a, se, cn, w_msg, w_self, b_msg, b_self):
    return pl.pallas_call(
        _stats_body,
        grid=(NGRID,),
        in_specs=[
            pl.BlockSpec((NB_STATS, XW), lambda i: (i, 0)),
            pl.BlockSpec((NB_STATS, XW), lambda i: (i, 0)),
            pl.BlockSpec((NB_STATS, EDGE), lambda i: (i, 0)),
            pl.BlockSpec((NB_STATS, EDGE), lambda i: (i, 0)),
            pl.BlockSpec((FEAT + EDGE, HID), lambda i: (0, 0)),
            pl.BlockSpec((FEAT, HID), lambda i: (0, 0)),
            pl.BlockSpec((1, HID), lambda i: (0, 0)),
            pl.BlockSpec((1, HID), lambda i: (0, 0)),
        ],
        out_specs=pl.BlockSpec((2 * T, HID), lambda i: (0, 0)),
        out_shape=jax.ShapeDtypeStruct((2 * T, HID), jnp.float32),
    )(x, a, se, cn, w_msg, w_self, b_msg, b_self)


def _final_body(xf_ref, af_ref, sf_ref, cf_ref, st_ref, wm_ref, ws_ref,
                bm_ref, bs_ref, g_ref, bb_ref, wih_ref, whh_ref, bih_ref,
                bhh_ref, wcls_ref, bcls_ref, o_ref):
    wx = wm_ref[0:FEAT, :]
    we = wm_ref[FEAT:FEAT + EDGE, :]
    ws = ws_ref[...]
    mean = st_ref[0:T, :] * (1.0 / N)
    var = st_ref[T:2 * T, :] * (1.0 / N) - mean * mean
    scale = g_ref[...] * lax.rsqrt(var + 1e-5)
    shift = bb_ref[...] - mean * scale
    cn = cf_ref[:, 0:1]
    basem = jnp.dot(sf_ref[...], we, preferred_element_type=jnp.float32)
    basem = basem + cn * bm_ref[...]
    rinv = 1.0 / jnp.maximum(cn, 1.0)
    h = jnp.zeros((B, HID), jnp.float32)
    c = jnp.zeros((B, HID), jnp.float32)
    for t in range(T):
        xt = xf_ref[:, FEAT * t:FEAT * (t + 1)]
        at = af_ref[:, FEAT * t:FEAT * (t + 1)]
        out = jnp.dot(xt, ws, preferred_element_type=jnp.float32)
        out = out + bs_ref[...]
        out = out + (jnp.dot(at, wx, preferred_element_type=jnp.float32)
                     + basem) * rinv
        ht = jnp.maximum(out * scale[t:t + 1, :] + shift[t:t + 1, :], 0.0)
        gates = (jnp.dot(ht, wih_ref[...], preferred_element_type=jnp.float32)
                 + bih_ref[...]
                 + jnp.dot(h, whh_ref[...], preferred_element_type=jnp.float32)
                 + bhh_ref[...])
        i_g = jax.nn.sigmoid(gates[:, 0 * HID:1 * HID])
        f_g = jax.nn.sigmoid(gates[:, 1 * HID:2 * HID])
        g_g = jnp.tanh(gates[:, 2 * HID:3 * HID])
        o_g = jax.nn.sigmoid(gates[:, 3 * HID:4 * HID])
        c = f_g * c + i_g * g_g
        h = o_g * jnp.tanh(c)
    o_ref[...] = jnp.dot(h, wcls_ref[...],
                         preferred_element_type=jnp.float32) + bcls_ref[...]


def _final(xf, af, sf, cf, stats, w_msg, w_self, b_msg, b_self, gam, bet,
           w_ih, w_hh, b_ih, b_hh, wcls_p, bcls_p):
    return pl.pallas_call(
        _final_body,
        out_shape=jax.ShapeDtypeStruct((B, HID), jnp.float32),
    )(xf, af, sf, cf, stats, w_msg, w_self, b_msg, b_self, gam, bet,
      w_ih, w_hh, b_ih, b_hh, wcls_p, bcls_p)


def kernel(x, edge_index, edge_attr, ptr, W_msg, b_msg, W_self, b_self,
           bn_gamma, bn_beta, W_ih, W_hh, b_ih, b_hh, W_cls, b_cls):
    src = edge_index[0]
    dst = edge_index[1]
    focal = ptr[:B]

    a, se, cn = _edge_pass(x, src, dst, edge_attr)
    xf, af, sf, cf = _focal_gather(focal, x, a, se, cn)

    b_msg2 = b_msg.reshape(1, HID)
    b_self2 = b_self.reshape(1, HID)
    stats = _stats(x, a, se, cn, W_msg, W_self, b_msg2, b_self2)

    gam = jnp.broadcast_to(bn_gamma.reshape(1, HID), (T, HID))
    bet = jnp.broadcast_to(bn_beta.reshape(1, HID), (T, HID))
    wcls_p = jnp.zeros((HID, HID), jnp.float32).at[:, :2].set(W_cls)
    bcls_p = jnp.zeros((1, HID), jnp.float32).at[0, :2].set(b_cls)
    out = _final(xf, af, sf, cf, stats, W_msg, W_self, b_msg2, b_self2,
                 gam, bet, W_ih, W_hh, b_ih.reshape(1, 4 * HID),
                 b_hh.reshape(1, 4 * HID), wcls_p, bcls_p)
    return out[:, :2]


# speedup vs baseline: 9.0817x; 1.0101x over previous
import functools

import jax
import jax.numpy as jnp
from jax import lax
from jax.experimental import pallas as pl
from jax.experimental.pallas import tpu as pltpu
from jax.experimental.pallas import tpu_sc as plsc

N = 50000
E = 800000
T = 4
FEAT = 64
HID = 128
EDGE = 16
B = 1024
XW = T * FEAT

NCORES = 2
NSUB = 16
CHUNK = 4480
NCHUNK = 12
NPASS = NCHUNK // NCORES
NPAD = NCHUNK * CHUNK
EPT = E // NSUB
EB = 2000
NBLK = EPT // EB
SUB = 16
NSLOT = 8
ROWS_PT = CHUNK // NSUB


def _edge_pass(x, src, dst, ea):
    mesh = plsc.VectorSubcoreMesh(core_axis_name="c", subcore_axis_name="s")

    @functools.partial(
        pl.kernel,
        out_type=(
            jax.ShapeDtypeStruct((NPAD, XW), jnp.float32),
            jax.ShapeDtypeStruct((NPAD, EDGE), jnp.float32),
            jax.ShapeDtypeStruct((NPAD, EDGE), jnp.float32),
        ),
        mesh=mesh,
        compiler_params=pltpu.CompilerParams(needs_layout_passes=False, use_tc_tiling_on_sc=False),
        scratch_types=dict(
            accX=pltpu.VMEM_SHARED((CHUNK + 16, XW), jnp.float32),
            accE=pltpu.VMEM_SHARED((CHUNK + 16, EDGE), jnp.float32),
            accC=pltpu.VMEM_SHARED((CHUNK + 16, EDGE), jnp.float32),
            src_blk=pltpu.VMEM((EB,), jnp.int32),
            dst_blk=pltpu.VMEM((EB,), jnp.int32),
            csrc=pltpu.VMEM((128, SUB), jnp.int32),
            cdst=pltpu.VMEM((128, SUB), jnp.int32),
            ceid=pltpu.VMEM((128, SUB), jnp.int32),
            stage_x=pltpu.VMEM((NSLOT, SUB, XW), jnp.float32),
            ea_st=pltpu.VMEM((NSLOT, SUB, EDGE), jnp.float32),
            ones_b=pltpu.VMEM((SUB, EDGE), jnp.float32),
            gxs=pltpu.SemaphoreType.DMA((NSLOT,)),
            ges=pltpu.SemaphoreType.DMA((NSLOT,)),
            scs=pltpu.SemaphoreType.DMA((NSLOT,)),
        ),
    )
    def edge_kernel(x_hbm, src_hbm, dst_hbm, ea_hbm, a_out, se_out, cn_out,
                    accX, accE, accC, src_blk, dst_blk, csrc, cdst, ceid,
                    stage_x, ea_st, ones_b, gxs, ges, scs):
        cid = lax.axis_index("c")
        sid = lax.axis_index("s")

        zero16 = jnp.zeros((16,), jnp.float32)
        one16 = jnp.ones((16,), jnp.float32)

        def init_row(i, _):
            ones_b[i, pl.ds(0, 16)] = one16
            return 0

        lax.fori_loop(0, SUB, init_row, 0)

        def pass_body(p, _):
            lo = (NCORES * p + cid) * CHUNK
            def zrow(i, _):
                for q in range(XW // 16):
                    stage_x[0, i, pl.ds(q * 16, 16)] = zero16
                ea_st[0, i, pl.ds(0, 16)] = zero16
                return 0

            lax.fori_loop(0, SUB, zrow, 0)
            r0 = sid * ROWS_PT
            zoff = 0
            for zn in (16,) * 17 + (8,):
                pltpu.sync_copy(stage_x.at[0].at[pl.ds(0, zn)],
                                accX.at[pl.ds(r0 + zoff, zn)])
                pltpu.sync_copy(ea_st.at[0].at[pl.ds(0, zn)],
                                accE.at[pl.ds(r0 + zoff, zn)])
                pltpu.sync_copy(ea_st.at[0].at[pl.ds(0, zn)],
                                accC.at[pl.ds(r0 + zoff, zn)])
                zoff += zn
            @pl.when(sid == 0)
            def _():
                pltpu.sync_copy(stage_x.at[0].at[pl.ds(0, 16)],
                                accX.at[pl.ds(CHUNK, 16)])
                pltpu.sync_copy(ea_st.at[0].at[pl.ds(0, 16)],
                                accE.at[pl.ds(CHUNK, 16)])
                pltpu.sync_copy(ea_st.at[0].at[pl.ds(0, 16)],
                                accC.at[pl.ds(CHUNK, 16)])
            plsc.subcore_barrier()

            def blk_body(bk, kin):
                base = sid * EPT + bk * EB
                pltpu.sync_copy(src_hbm.at[pl.ds(base, EB)], src_blk)
                pltpu.sync_copy(dst_hbm.at[pl.ds(base, EB)], dst_blk)

                def filt(i, k):
                    d = dst_blk[pl.ds(i * 16, 16)]
                    s = src_blk[pl.ds(i * 16, 16)]
                    m = (d >= lo) & (d < lo + CHUNK)
                    mi = m.astype(jnp.int32)
                    pos = k + plsc.cumsum(mi) - 1
                    pr = lax.shift_right_logical(pos, 4)
                    pc = lax.bitwise_and(pos, 15)
                    plsc.store_scatter(cdst, [pr, pc], d - lo, mask=m)
                    plsc.store_scatter(csrc, [pr, pc], s, mask=m)
                    ev = base + i * 16 + lax.iota(jnp.int32, 16)
                    plsc.store_scatter(ceid, [pr, pc], ev, mask=m)
                    return k + jnp.sum(mi)

                k = lax.fori_loop(0, EB // 16, filt, kin, unroll=4)
                nsub = k // SUB

                def issue(jj, sl):
                    pltpu.async_copy(x_hbm.at[csrc.at[jj]],
                                     stage_x.at[sl], gxs.at[sl])
                    pltpu.async_copy(ea_hbm.at[ceid.at[jj]],
                                     ea_st.at[sl], ges.at[sl])

                def drain(sl):
                    pltpu.make_async_copy(stage_x.at[0],
                                          accX.at[cdst.at[0]],
                                          scs.at[sl]).wait()
                    pltpu.make_async_copy(ea_st.at[0],
                                          accE.at[cdst.at[0]],
                                          scs.at[sl]).wait()
                    pltpu.make_async_copy(ones_b, accC.at[cdst.at[0]],
                                          scs.at[sl]).wait()

                for jj in range(NSLOT - 1):
                    @pl.when(jj < nsub)
                    def _(jj=jj):
                        issue(jj, jj)

                def sub(j, _):
                    s = j & 7
                    pltpu.make_async_copy(x_hbm.at[csrc.at[j]],
                                          stage_x.at[s], gxs.at[s]).wait()
                    pltpu.make_async_copy(ea_hbm.at[ceid.at[j]],
                                          ea_st.at[s], ges.at[s]).wait()
                    @pl.when(j + NSLOT - 1 < nsub)
                    def _():
                        @pl.when(j >= 1)
                        def _():
                            drain((s + 7) & 7)
                        issue(j + NSLOT - 1, (s + 7) & 7)
                    pltpu.async_copy(stage_x.at[s], accX.at[cdst.at[j]],
                                     scs.at[s], add=True)
                    pltpu.async_copy(ea_st.at[s], accE.at[cdst.at[j]],
                                     scs.at[s], add=True)
                    pltpu.async_copy(ones_b, accC.at[cdst.at[j]],
                                     scs.at[s], add=True)
                    return 0

                lax.fori_loop(0, nsub, sub, 0)
                for ss in range(NSLOT):
                    @pl.when(ss < nsub)
                    def _(ss=ss):
                        drain(ss)
                for q in range(SUB // 16):
                    cdst[0, pl.ds(q * 16, 16)] = cdst[nsub, pl.ds(q * 16, 16)]
                    csrc[0, pl.ds(q * 16, 16)] = csrc[nsub, pl.ds(q * 16, 16)]
                    ceid[0, pl.ds(q * 16, 16)] = ceid[nsub, pl.ds(q * 16, 16)]
                return k & (SUB - 1)

            kfl = lax.fori_loop(0, NBLK, blk_body, 0)
            @pl.when(kfl > 0)
            def _():
                dummy_d = jnp.full((16,), CHUNK, jnp.int32)
                zero_i = jnp.zeros((16,), jnp.int32)
                for q in range(SUB // 16):
                    pos = kfl + q * 16 + lax.iota(jnp.int32, 16)
                    pr = lax.shift_right_logical(pos, 4)
                    pc = lax.bitwise_and(pos, 15)
                    plsc.store_scatter(cdst, [pr, pc], dummy_d)
                    plsc.store_scatter(csrc, [pr, pc], zero_i)
                    plsc.store_scatter(ceid, [pr, pc], zero_i)
                pltpu.async_copy(x_hbm.at[csrc.at[0]], stage_x.at[0],
                                 gxs.at[0])
                pltpu.async_copy(ea_hbm.at[ceid.at[0]], ea_st.at[0],
                                 ges.at[0])
                pltpu.make_async_copy(x_hbm.at[csrc.at[0]], stage_x.at[0],
                                      gxs.at[0]).wait()
                pltpu.make_async_copy(ea_hbm.at[ceid.at[0]], ea_st.at[0],
                                      ges.at[0]).wait()
                pltpu.sync_copy(stage_x.at[0], accX.at[cdst.at[0]],
                                add=True)
                pltpu.sync_copy(ea_st.at[0], accE.at[cdst.at[0]],
                                add=True)
                pltpu.sync_copy(ones_b, accC.at[cdst.at[0]], add=True)
            plsc.subcore_barrier()
            pltpu.sync_copy(accX.at[pl.ds(r0, ROWS_PT)],
                            a_out.at[pl.ds(lo + r0, ROWS_PT)])
            pltpu.sync_copy(accE.at[pl.ds(r0, ROWS_PT)],
                            se_out.at[pl.ds(lo + r0, ROWS_PT)])
            pltpu.sync_copy(accC.at[pl.ds(r0, ROWS_PT)],
                            cn_out.at[pl.ds(lo + r0, ROWS_PT)])
            return 0

        lax.fori_loop(0, NPASS, pass_body, 0)

    return edge_kernel(x, src, dst, ea)


def _focal_gather(focal, x, a, se, cn):
    mesh = plsc.VectorSubcoreMesh(core_axis_name="c", subcore_axis_name="s")
    RPW = B // (NCORES * NSUB)

    @functools.partial(
        pl.kernel,
        out_type=(
            jax.ShapeDtypeStruct((B, XW), jnp.float32),
            jax.ShapeDtypeStruct((B, XW), jnp.float32),
            jax.ShapeDtypeStruct((B, EDGE), jnp.float32),
            jax.ShapeDtypeStruct((B, EDGE), jnp.float32),
        ),
        mesh=mesh,
        compiler_params=pltpu.CompilerParams(needs_layout_passes=False, use_tc_tiling_on_sc=False),
        scratch_types=dict(
            pidx=pltpu.VMEM((RPW,), jnp.int32),
            bufx=pltpu.VMEM((RPW, XW), jnp.float32),
            bufa=pltpu.VMEM((RPW, XW), jnp.float32),
            bufs=pltpu.VMEM((RPW, EDGE), jnp.float32),
            bufc=pltpu.VMEM((RPW, EDGE), jnp.float32),
            sem=pltpu.SemaphoreType.DMA,
        ),
    )
    def focal_kernel(f_hbm, x_hbm, a_hbm, se_hbm, cn_hbm,
                     xf, af, sf, cf, pidx, bufx, bufa, bufs, bufc, sem):
        wid = lax.axis_index("s") * NCORES + lax.axis_index("c")
        base = wid * RPW
        pltpu.sync_copy(f_hbm.at[pl.ds(base, RPW)], pidx)
        pltpu.async_copy(x_hbm.at[pidx], bufx, sem).wait()
        pltpu.sync_copy(bufx, xf.at[pl.ds(base, RPW)])
        pltpu.async_copy(a_hbm.at[pidx], bufa, sem).wait()
        pltpu.sync_copy(bufa, af.at[pl.ds(base, RPW)])
        pltpu.async_copy(se_hbm.at[pidx], bufs, sem).wait()
        pltpu.sync_copy(bufs, sf.at[pl.ds(base, RPW)])
        pltpu.async_copy(cn_hbm.at[pidx], bufc, sem).wait()
        pltpu.sync_copy(bufc, cf.at[pl.ds(base, RPW)])

    return focal_kernel(focal, x, a, se, cn)


NB_STATS = 1000
NGRID = N // NB_STATS


def _stats_body(x_ref, a_ref, se_ref, cn_ref, wm_ref, ws_ref, bm_ref, bs_ref,
                o_ref):
    pc = pl.program_id(0)

    @pl.when(pc == 0)
    def _():
        o_ref[...] = jnp.zeros_like(o_ref)

    wx = wm_ref[0:FEAT, :]
    we = wm_ref[FEAT:FEAT + EDGE, :]
    ws = ws_ref[...]
    cn = cn_ref[:, 0:1]
    basem = jnp.dot(se_ref[...], we, preferred_element_type=jnp.float32)
    basem = basem + cn * bm_ref[...]
    rinv = 1.0 / jnp.maximum(cn, 1.0)
    rows = []
    rows2 = []
    for t in range(T):
        xt = x_ref[:, FEAT * t:FEAT * (t + 1)]
        at = a_ref[:, FEAT * t:FEAT * (t + 1)]
        out = jnp.dot(xt, ws, preferred_element_type=jnp.float32)
        out = out + bs_ref[...]
        out = out + (jnp.dot(at, wx, preferred_element_type=jnp.float32)
                     + basem) * rinv
        rows.append(jnp.sum(out, axis=0, keepdims=True))
        rows2.append(jnp.sum(out * out, axis=0, keepdims=True))
    o_ref[...] += jnp.concatenate(rows + rows2, axis=0)


def _stats(x, a, se, cn, w_msg, w_self, b_msg, b_self):
    return pl.pallas_call(
        _stats_body,
        grid=(NGRID,),
        in_specs=[
            pl.BlockSpec((NB_STATS, XW), lambda i: (i, 0)),
            pl.BlockSpec((NB_STATS, XW), lambda i: (i, 0)),
            pl.BlockSpec((NB_STATS, EDGE), lambda i: (i, 0)),
            pl.BlockSpec((NB_STATS, EDGE), lambda i: (i, 0)),
            pl.BlockSpec((FEAT + EDGE, HID), lambda i: (0, 0)),
            pl.BlockSpec((FEAT, HID), lambda i: (0, 0)),
            pl.BlockSpec((1, HID), lambda i: (0, 0)),
            pl.BlockSpec((1, HID), lambda i: (0, 0)),
        ],
        out_specs=pl.BlockSpec((2 * T, HID), lambda i: (0, 0)),
        out_shape=jax.ShapeDtypeStruct((2 * T, HID), jnp.float32),
    )(x, a, se, cn, w_msg, w_self, b_msg, b_self)


def _final_body(xf_ref, af_ref, sf_ref, cf_ref, st_ref, wm_ref, ws_ref,
                bm_ref, bs_ref, g_ref, bb_ref, wih_ref, whh_ref, bih_ref,
                bhh_ref, wcls_ref, bcls_ref, o_ref):
    wx = wm_ref[0:FEAT, :]
    we = wm_ref[FEAT:FEAT + EDGE, :]
    ws = ws_ref[...]
    mean = st_ref[0:T, :] * (1.0 / N)
    var = st_ref[T:2 * T, :] * (1.0 / N) - mean * mean
    scale = g_ref[...] * lax.rsqrt(var + 1e-5)
    shift = bb_ref[...] - mean * scale
    cn = cf_ref[:, 0:1]
    basem = jnp.dot(sf_ref[...], we, preferred_element_type=jnp.float32)
    basem = basem + cn * bm_ref[...]
    rinv = 1.0 / jnp.maximum(cn, 1.0)
    h = jnp.zeros((B, HID), jnp.float32)
    c = jnp.zeros((B, HID), jnp.float32)
    for t in range(T):
        xt = xf_ref[:, FEAT * t:FEAT * (t + 1)]
        at = af_ref[:, FEAT * t:FEAT * (t + 1)]
        out = jnp.dot(xt, ws, preferred_element_type=jnp.float32)
        out = out + bs_ref[...]
        out = out + (jnp.dot(at, wx, preferred_element_type=jnp.float32)
                     + basem) * rinv
        ht = jnp.maximum(out * scale[t:t + 1, :] + shift[t:t + 1, :], 0.0)
        gates = (jnp.dot(ht, wih_ref[...], preferred_element_type=jnp.float32)
                 + bih_ref[...]
                 + jnp.dot(h, whh_ref[...], preferred_element_type=jnp.float32)
                 + bhh_ref[...])
        i_g = jax.nn.sigmoid(gates[:, 0 * HID:1 * HID])
        f_g = jax.nn.sigmoid(gates[:, 1 * HID:2 * HID])
        g_g = jnp.tanh(gates[:, 2 * HID:3 * HID])
        o_g = jax.nn.sigmoid(gates[:, 3 * HID:4 * HID])
        c = f_g * c + i_g * g_g
        h = o_g * jnp.tanh(c)
    o_ref[...] = jnp.dot(h, wcls_ref[...],
                         preferred_element_type=jnp.float32) + bcls_ref[...]


def _final(xf, af, sf, cf, stats, w_msg, w_self, b_msg, b_self, gam, bet,
           w_ih, w_hh, b_ih, b_hh, wcls_p, bcls_p):
    return pl.pallas_call(
        _final_body,
        out_shape=jax.ShapeDtypeStruct((B, HID), jnp.float32),
    )(xf, af, sf, cf, stats, w_msg, w_self, b_msg, b_self, gam, bet,
      w_ih, w_hh, b_ih, b_hh, wcls_p, bcls_p)


def kernel(x, edge_index, edge_attr, ptr, W_msg, b_msg, W_self, b_self,
           bn_gamma, bn_beta, W_ih, W_hh, b_ih, b_hh, W_cls, b_cls):
    src = edge_index[0]
    dst = edge_index[1]
    focal = ptr[:B]

    a, se, cn = _edge_pass(x, src, dst, edge_attr)
    xf, af, sf, cf = _focal_gather(focal, x, a, se, cn)

    b_msg2 = b_msg.reshape(1, HID)
    b_self2 = b_self.reshape(1, HID)
    stats = _stats(x, a, se, cn, W_msg, W_self, b_msg2, b_self2)

    gam = jnp.broadcast_to(bn_gamma.reshape(1, HID), (T, HID))
    bet = jnp.broadcast_to(bn_beta.reshape(1, HID), (T, HID))
    wcls_p = jnp.zeros((HID, HID), jnp.float32).at[:, :2].set(W_cls)
    bcls_p = jnp.zeros((1, HID), jnp.float32).at[0, :2].set(b_cls)
    out = _final(xf, af, sf, cf, stats, W_msg, W_self, b_msg2, b_self2,
                 gam, bet, W_ih, W_hh, b_ih.reshape(1, 4 * HID),
                 b_hh.reshape(1, 4 * HID), wcls_p, bcls_p)
    return out[:, :2]
